# baseline (speedup 1.0000x reference)
import functools

import jax
import jax.numpy as jnp
from jax.experimental import pallas as pl
from jax.experimental.pallas import tpu as pltpu

_EPS = 1e-5
_VMEM_LIMIT = 48 * 1024 * 1024


def _lane_shift(x, d):
    S = x.shape[-1]
    d = d % S
    if d == 0:
        return x
    return jnp.concatenate([x[:, d:], x[:, :d]], axis=-1)


def _gather3x3(x, H, W):
    S = H * W
    col = jax.lax.broadcasted_iota(jnp.int32, (1, S), 1)
    yy = col // W
    xx = col % W
    parts = []
    for wy in range(3):
        for wx in range(3):
            dy, dx = wy - 1, wx - 1
            shifted = _lane_shift(x, dy * W + dx)
            valid = ((yy + dy >= 0) & (yy + dy < H) &
                     (xx + dx >= 0) & (xx + dx < W))
            parts.append(jnp.where(valid, shifted, 0.0))
    return jnp.concatenate(parts, axis=0)


def _gen_noise(x, noise, upper, lower):
    S = x.shape[-1]
    cmax = jnp.max(x, axis=-1, keepdims=True)
    s = jnp.sum(x, axis=-1, keepdims=True)
    q = jnp.sum(x * x, axis=-1, keepdims=True)
    mean = s * (1.0 / S)
    var = jnp.maximum((q - S * mean * mean) * (1.0 / (S - 1)), 0.0)
    std = jnp.sqrt(var)
    clone = jnp.where(x < -cmax * (1.0 / lower), 0.0, x)
    clone = jnp.where(clone > cmax * (1.0 / upper), 0.0, clone)
    return x + clone * (noise * std)


def _conv3x3(a, wmat, bias, H, W):
    patches = _gather3x3(a.astype(jnp.bfloat16), H, W)
    y = jnp.dot(wmat, patches, preferred_element_type=jnp.float32)
    return y + bias


def _row_shift(x, d):
    if d % x.shape[0] == 0:
        return x
    return jnp.concatenate([x[d:], x[:d]], axis=0)


def _bn_cols_folded(y, g, b, cout, eps):
    m = y.shape[0]
    s = jnp.sum(y, axis=0, keepdims=True)
    q = jnp.sum(y * y, axis=0, keepdims=True)
    sc = s[:, 0:cout] + s[:, cout:2 * cout] + s[:, 2 * cout:3 * cout] \
        + s[:, 3 * cout:4 * cout]
    qc = q[:, 0:cout] + q[:, cout:2 * cout] + q[:, 2 * cout:3 * cout] \
        + q[:, 3 * cout:4 * cout]
    cnt = 4.0 * m
    mean = sc * (1.0 / cnt)
    var = jnp.maximum(qc * (1.0 / cnt) - mean * mean, 0.0)
    scale = g * jax.lax.rsqrt(var + eps)
    shift = b - mean * scale
    scale4 = jnp.concatenate([scale] * 4, axis=1)
    shift4 = jnp.concatenate([shift] * 4, axis=1)
    return jnp.maximum(y * scale4 + shift4, 0.0)


def _head_kernel(a_ref, w1_ref, g1_ref, b1_ref, w2_ref, g2_ref, b2_ref,
                 w3_ref, g3_ref, b3_ref, o_ref, *, eps):
    y = jnp.dot(a_ref[...], w1_ref[...], preferred_element_type=jnp.float32)
    m = y.shape[0]
    mean = jnp.sum(y, axis=0, keepdims=True) * (1.0 / m)
    var = jnp.maximum(jnp.sum(y * y, axis=0, keepdims=True) * (1.0 / m)
                      - mean * mean, 0.0)
    scale = g1_ref[...] * jax.lax.rsqrt(var + eps)
    shift = b1_ref[...] - mean * scale
    h1 = jnp.maximum(y * scale + shift, 0.0).astype(jnp.bfloat16)

    row = jax.lax.broadcasted_iota(jnp.int32, (m, 1), 0)
    yy = (row % 16) // 4
    xx = row % 4
    parts = []
    for dy in (-1, 0, 1):
        for dx in (-1, 0, 1):
            shifted = _row_shift(h1, dy * 4 + dx)
            valid = ((yy + dy >= 0) & (yy + dy < 4) &
                     (xx + dx >= 0) & (xx + dx < 4))
            parts.append(jnp.where(valid, shifted, 0.0))
    patches2 = jnp.concatenate(parts, axis=1)
    y2 = jnp.dot(patches2, w2_ref[...], preferred_element_type=jnp.float32)
    h2 = _bn_cols_folded(y2, g2_ref[...], b2_ref[...], 128, eps) \
        .astype(jnp.bfloat16)

    aa = (row % 16) // 4
    bb = row % 4
    y3s = {}
    s_acc = None
    q_acc = None
    for ry in range(2):
        for rx in range(2):
            p3 = []
            for dy in (-1, 0, 1):
                for dx in (-1, 0, 1):
                    ty, tx = ry + dy, rx + dx
                    qy, qx = ty % 2, tx % 2
                    sy, sx = (ty - qy) // 2, (tx - qx) // 2
                    g = (2 * qy + qx) * 128
                    blk = _row_shift(h2[:, g:g + 128], sy * 4 + sx)
                    valid = ((aa + sy >= 0) & (aa + sy < 4) &
                             (bb + sx >= 0) & (bb + sx < 4))
                    p3.append(jnp.where(valid, blk, 0.0))
            patches3 = jnp.concatenate(p3, axis=1)
            y3 = jnp.dot(patches3, w3_ref[...],
                         preferred_element_type=jnp.float32)
            y3s[(ry, rx)] = y3
            s = jnp.sum(y3, axis=0, keepdims=True)
            q = jnp.sum(y3 * y3, axis=0, keepdims=True)
            s_acc = s if s_acc is None else s_acc + s
            q_acc = q if q_acc is None else q_acc + q
    cout = 64
    sc = s_acc[:, 0:cout] + s_acc[:, cout:2 * cout] \
        + s_acc[:, 2 * cout:3 * cout] + s_acc[:, 3 * cout:4 * cout]
    qc = q_acc[:, 0:cout] + q_acc[:, cout:2 * cout] \
        + q_acc[:, 2 * cout:3 * cout] + q_acc[:, 3 * cout:4 * cout]
    cnt = 16.0 * m
    mean = sc * (1.0 / cnt)
    var = jnp.maximum(qc * (1.0 / cnt) - mean * mean, 0.0)
    scale = g3_ref[...] * jax.lax.rsqrt(var + eps)
    shift = b3_ref[...] - mean * scale
    scale4 = jnp.concatenate([scale] * 4, axis=1)
    shift4 = jnp.concatenate([shift] * 4, axis=1)
    for p, (ry, rx) in enumerate([(0, 0), (0, 1), (1, 0), (1, 1)]):
        o_ref[p] = jnp.maximum(y3s[(ry, rx)] * scale4 + shift4,
                               0.0).astype(o_ref.dtype)


def _planar_ct_bn_kernel(x_ref, w_ref, g_ref, b_ref, o_ref, *, H, W,
                         phase_major, eps):
    n_batch = x_ref.shape[0]
    S = H * W
    w = w_ref[...]
    ys, s_acc, q_acc = [], None, None
    for n in range(n_batch):
        patches = _gather3x3(x_ref[n], H, W)
        y = jnp.dot(w, patches, preferred_element_type=jnp.float32)
        ys.append(y)
        s_n = jnp.sum(y, axis=1, keepdims=True)
        q_n = jnp.sum(y * y, axis=1, keepdims=True)
        s_acc = s_n if s_acc is None else s_acc + s_n
        q_acc = q_n if q_acc is None else q_acc + q_n
    r = s_acc.shape[0]
    ri = jax.lax.broadcasted_iota(jnp.int32, (r, r), 0)
    cj = jax.lax.broadcasted_iota(jnp.int32, (r, r), 1)
    if phase_major:
        fold = ((ri % (r // 4)) == (cj % (r // 4))).astype(jnp.float32)
    else:
        fold = ((ri // 4) == (cj // 4)).astype(jnp.float32)
    stats = jnp.dot(fold, jnp.concatenate([s_acc, q_acc], axis=1),
                    preferred_element_type=jnp.float32)
    cnt = float(n_batch * 4 * S)
    mean = stats[:, 0:1] * (1.0 / cnt)
    var = jnp.maximum(stats[:, 1:2] * (1.0 / cnt) - mean * mean, 0.0)
    scale = g_ref[...] * jax.lax.rsqrt(var + eps)
    shift = b_ref[...] - mean * scale
    for n in range(n_batch):
        o_ref[n] = jnp.maximum(ys[n] * scale + shift, 0.0).astype(o_ref.dtype)


def _subphase_ct_bn_kernel(x_ref, w_ref, g_ref, b_ref, o_ref, *, H, W,
                           phase_major, eps):
    n_batch = x_ref.shape[0]
    C = x_ref.shape[1] // 4
    S = H * W
    w = w_ref[...]
    col = jax.lax.broadcasted_iota(jnp.int32, (1, S), 1)
    aa = col // W
    bb = col % W
    ys, s_acc, q_acc = [], None, None
    for n in range(n_batch):
        x = x_ref[n]
        pieces = []
        for py in range(2):
            for px in range(2):
                parts = []
                for dy in (-1, 0, 1):
                    for dx in (-1, 0, 1):
                        ty, tx = py + dy, px + dx
                        qy, qx = ty % 2, tx % 2
                        sy, sx = (ty - qy) // 2, (tx - qx) // 2
                        blk = x[(2 * qy + qx) * C:(2 * qy + qx + 1) * C]
                        shifted = _lane_shift(blk, sy * W + sx)
                        valid = ((aa + sy >= 0) & (aa + sy < H) &
                                 (bb + sx >= 0) & (bb + sx < W))
                        parts.append(jnp.where(valid, shifted, 0.0))
                patches = jnp.concatenate(parts, axis=0)
                y = jnp.dot(w, patches,
                            preferred_element_type=jnp.float32)
                pieces.append(y)
                s_p = jnp.sum(y, axis=1, keepdims=True)
                q_p = jnp.sum(y * y, axis=1, keepdims=True)
                s_acc = s_p if s_acc is None else s_acc + s_p
                q_acc = q_p if q_acc is None else q_acc + q_p
        ys.append(pieces)
    r = s_acc.shape[0]
    ri = jax.lax.broadcasted_iota(jnp.int32, (r, r), 0)
    cj = jax.lax.broadcasted_iota(jnp.int32, (r, r), 1)
    if phase_major:
        fold = ((ri % (r // 4)) == (cj % (r // 4))).astype(jnp.float32)
    else:
        fold = ((ri // 4) == (cj // 4)).astype(jnp.float32)
    stats = jnp.dot(fold, jnp.concatenate([s_acc, q_acc], axis=1),
                    preferred_element_type=jnp.float32)
    cnt = float(n_batch * 16 * S)
    mean = stats[:, 0:1] * (1.0 / cnt)
    var = jnp.maximum(stats[:, 1:2] * (1.0 / cnt) - mean * mean, 0.0)
    scale = g_ref[...] * jax.lax.rsqrt(var + eps)
    shift = b_ref[...] - mean * scale
    for n in range(n_batch):
        for p in range(4):
            o_ref[n, p] = jnp.maximum(ys[n][p] * scale + shift,
                                      0.0).astype(o_ref.dtype)


def _ct6_kernel(x_ref, w_ref, o_ref, *, H, W):
    S = H * W
    col = jax.lax.broadcasted_iota(jnp.int32, (1, S), 1)
    aa = col // W
    bb = col % W
    C = x_ref.shape[2] // 4
    w = w_ref[...]
    for py in range(2):
        for px in range(2):
            for py2 in range(2):
                for px2 in range(2):
                    parts = []
                    for dy in (-1, 0, 1):
                        for dx in (-1, 0, 1):
                            ty = 2 * py + py2 + dy
                            tx = 2 * px + px2 + dx
                            qy, qx = ty % 4, tx % 4
                            sy, sx = (ty - qy) // 4, (tx - qx) // 4
                            blk = x_ref[0, 2 * (qy // 2) + (qx // 2),
                                        (2 * (qy % 2) + (qx % 2)) * C:
                                        (2 * (qy % 2) + (qx % 2) + 1) * C]
                            shifted = _lane_shift(blk, sy * W + sx)
                            valid = ((aa + sy >= 0) & (aa + sy < H) &
                                     (bb + sx >= 0) & (bb + sx < W))
                            parts.append(jnp.where(valid, shifted, 0.0))
                    patches = jnp.concatenate(parts, axis=0)
                    y = jnp.dot(w, patches,
                                preferred_element_type=jnp.float32)
                    o_ref[0, 2 * py + px, 2 * py2 + px2] = \
                        y.astype(o_ref.dtype)


def _tail_kernel(x_ref, n1_ref, n2_ref, w1_ref, b1_ref, w2_ref, b2_ref,
                 w3_ref, b3_ref, w4_ref, b4_ref, o_ref, *, H, W, nc,
                 upper, lower):
    S = H * W
    c2 = n2_ref.shape[1]
    a = x_ref[0].astype(jnp.float32)
    n1 = n1_ref[0].astype(jnp.float32)
    a = _gen_noise(a, n1, upper, lower)
    a = _conv3x3(a, w1_ref[...], b1_ref[...], H, W)
    a = _conv3x3(a, w2_ref[...], b2_ref[...], H, W)
    n2 = jnp.concatenate(
        [n2_ref[0].astype(jnp.float32),
         jnp.zeros((a.shape[0] - c2, S), jnp.float32)], axis=0)
    a = _gen_noise(a, n2, upper, lower)
    a = _conv3x3(a, w3_ref[...], b3_ref[...], H, W)
    a = _conv3x3(a, w4_ref[...], b4_ref[...], H, W)
    o_ref[0] = jnp.tanh(a[:nc, :])


def _head(a1, w1, g1, b1, w2, g2, b2, w3, g3, b3):
    M = a1.shape[0]

    def rep(arr):
        return pl.BlockSpec(arr.shape, lambda i, nd=arr.ndim: (0,) * nd)

    return pl.pallas_call(
        functools.partial(_head_kernel, eps=_EPS),
        out_shape=jax.ShapeDtypeStruct((4, M, 256), jnp.bfloat16),
        grid=(1,),
        in_specs=[rep(a1), rep(w1), rep(g1), rep(b1),
                  rep(w2), rep(g2), rep(b2), rep(w3), rep(g3), rep(b3)],
        out_specs=pl.BlockSpec((4, M, 256), lambda i: (0, 0, 0)),
        compiler_params=pltpu.CompilerParams(
            dimension_semantics=("arbitrary",),
            vmem_limit_bytes=_VMEM_LIMIT),
    )(a1.astype(jnp.bfloat16), w1, g1, b1, w2, g2, b2, w3, g3, b3)


def _planar_ct_bn_relu(x, wpl, gamma_rows, beta_rows, *, H, W,
                       phase_major=False):
    N, Cin, S = x.shape
    R, K = wpl.shape
    return pl.pallas_call(
        functools.partial(_planar_ct_bn_kernel, H=H, W=W,
                          phase_major=phase_major, eps=_EPS),
        out_shape=jax.ShapeDtypeStruct((N, R, S), jnp.bfloat16),
        grid=(1,),
        in_specs=[pl.BlockSpec((N, Cin, S), lambda i: (0, 0, 0)),
                  pl.BlockSpec((R, K), lambda i: (0, 0)),
                  pl.BlockSpec((R, 1), lambda i: (0, 0)),
                  pl.BlockSpec((R, 1), lambda i: (0, 0))],
        out_specs=pl.BlockSpec((N, R, S), lambda i: (0, 0, 0)),
        compiler_params=pltpu.CompilerParams(
            dimension_semantics=("arbitrary",),
            vmem_limit_bytes=_VMEM_LIMIT),
    )(x, wpl, gamma_rows, beta_rows)


def _subphase_ct_bn_relu(x, wpl, gamma_rows, beta_rows, *, H, W,
                         phase_major=False):
    N, C4, S = x.shape
    R, K = wpl.shape
    return pl.pallas_call(
        functools.partial(_subphase_ct_bn_kernel, H=H, W=W,
                          phase_major=phase_major, eps=_EPS),
        out_shape=jax.ShapeDtypeStruct((N, 4, R, S), jnp.bfloat16),
        grid=(1,),
        in_specs=[pl.BlockSpec((N, C4, S), lambda i: (0, 0, 0)),
                  pl.BlockSpec((R, K), lambda i: (0, 0)),
                  pl.BlockSpec((R, 1), lambda i: (0, 0)),
                  pl.BlockSpec((R, 1), lambda i: (0, 0))],
        out_specs=pl.BlockSpec((N, 4, R, S), lambda i: (0, 0, 0, 0)),
        compiler_params=pltpu.CompilerParams(
            dimension_semantics=("arbitrary",),
            vmem_limit_bytes=_VMEM_LIMIT),
    )(x, wpl, gamma_rows, beta_rows)


def _ct6_phase(x, wpl, *, H, W):
    N, P4, R5, S = x.shape
    R, K = wpl.shape
    return pl.pallas_call(
        functools.partial(_ct6_kernel, H=H, W=W),
        out_shape=jax.ShapeDtypeStruct((N, 4, 4, R, S), jnp.bfloat16),
        grid=(N,),
        in_specs=[pl.BlockSpec((1, P4, R5, S), lambda n: (n, 0, 0, 0)),
                  pl.BlockSpec((R, K), lambda n: (0, 0))],
        out_specs=pl.BlockSpec((1, 4, 4, R, S), lambda n: (n, 0, 0, 0, 0)),
        compiler_params=pltpu.CompilerParams(
            dimension_semantics=("parallel",),
            vmem_limit_bytes=_VMEM_LIMIT),
    )(x, wpl)


def _tail(act, n1, n2, weights, *, nc, H, W, upper=4.0, lower=2.0):
    N, C0, S = act.shape
    c2 = n2.shape[1]
    w1, b1, w2, b2, w3, b3, w4, b4 = weights

    def rep_spec(arr):
        nd = arr.ndim
        return pl.BlockSpec(arr.shape, lambda n, nd=nd: (0,) * nd)

    return pl.pallas_call(
        functools.partial(_tail_kernel, H=H, W=W, nc=nc,
                          upper=upper, lower=lower),
        out_shape=jax.ShapeDtypeStruct((N, nc, S), jnp.float32),
        grid=(N,),
        in_specs=[pl.BlockSpec((1, C0, S), lambda n: (n, 0, 0)),
                  pl.BlockSpec((1, C0, S), lambda n: (n, 0, 0)),
                  pl.BlockSpec((1, c2, S), lambda n: (n, 0, 0)),
                  rep_spec(w1), rep_spec(b1), rep_spec(w2), rep_spec(b2),
                  rep_spec(w3), rep_spec(b3), rep_spec(w4), rep_spec(b4)],
        out_specs=pl.BlockSpec((1, nc, S), lambda n: (n, 0, 0)),
        compiler_params=pltpu.CompilerParams(
            dimension_semantics=("parallel",),
            vmem_limit_bytes=_VMEM_LIMIT),
    )(act, n1, n2, w1, b1, w2, b2, w3, b3, w4, b4)


def kernel(m1, m2, m3, m4, m5, m6,
           g1, b1, g2, b2, g3, b3, g4, b4, g5, b5,
           c1_w, c1_b, c2_w, c2_b, c3_w, c3_b, c4_w, c4_b,
           x, noise1, noise2):
    nc, ngf = 1, 16
    N, nz = x.shape[0], x.shape[1]
    z = x.reshape(N, nz).astype(jnp.bfloat16)

    eye16 = jnp.eye(16, dtype=z.dtype)
    a1 = (eye16[None, :, :, None] * z[:, None, None, :]).reshape(
        N * 16, 16 * nz)
    y3 = _head(a1, m1, g1, b1, m2, g2, b2, m3, g3, b3)
    hp = (y3.reshape(2, 2, N, 4, 4, 2, 2, 64)
          .transpose(2, 7, 3, 0, 5, 4, 1, 6)
          .reshape(N, ngf * 4, 256))

    m4pm = m4.reshape(ngf * 2, 4, m4.shape[1]).transpose(1, 0, 2) \
        .reshape(ngf * 8, m4.shape[1])
    g4pm = g4.reshape(ngf * 2, 4).transpose(1, 0).reshape(ngf * 8, 1)
    b4pm = b4.reshape(ngf * 2, 4).transpose(1, 0).reshape(ngf * 8, 1)
    y = _planar_ct_bn_relu(hp, m4pm, g4pm, b4pm, H=16, W=16,
                           phase_major=True)
    m5pm = m5.reshape(ngf, 4, m5.shape[1]).transpose(1, 0, 2) \
        .reshape(ngf * 4, m5.shape[1])
    g5pm = g5.reshape(ngf, 4).transpose(1, 0).reshape(ngf * 4, 1)
    b5pm = b5.reshape(ngf, 4).transpose(1, 0).reshape(ngf * 4, 1)
    y5p = _subphase_ct_bn_relu(y, m5pm, g5pm, b5pm, H=16, W=16,
                               phase_major=True)

    y6 = _ct6_phase(y5p, m6, H=16, W=16)
    act = (y6.reshape(N, 2, 2, 2, 2, 8, 2, 2, 16, 16)
           .transpose(0, 5, 8, 1, 3, 6, 9, 2, 4, 7)
           .reshape(N, ngf // 2, 128 * 128))

    S = 128 * 128
    n1 = noise1.reshape(N, ngf // 2, S)
    n2 = noise2.reshape(N, noise2.shape[1], S)
    weights = (c1_w, c1_b, c2_w, c2_b, c3_w, c3_b, c4_w, c4_b)
    out = _tail(act, n1, n2, weights, nc=nc, H=128, W=128)
    return out.reshape(N, nc, 128, 128)


# speedup vs baseline: 1.0797x; 1.0797x over previous
import functools

import jax
import jax.numpy as jnp
from jax.experimental import pallas as pl
from jax.experimental.pallas import tpu as pltpu

_EPS = 1e-5
_VMEM_LIMIT = 48 * 1024 * 1024


def _lane_shift(x, d):
    S = x.shape[-1]
    d = d % S
    if d == 0:
        return x
    return jnp.concatenate([x[:, d:], x[:, :d]], axis=-1)


def _gather3x3(x, H, W):
    S = H * W
    col = jax.lax.broadcasted_iota(jnp.int32, (1, S), 1)
    yy = col // W
    xx = col % W
    parts = []
    for wy in range(3):
        for wx in range(3):
            dy, dx = wy - 1, wx - 1
            shifted = _lane_shift(x, dy * W + dx)
            valid = ((yy + dy >= 0) & (yy + dy < H) &
                     (xx + dx >= 0) & (xx + dx < W))
            parts.append(jnp.where(valid, shifted, 0.0))
    return jnp.concatenate(parts, axis=0)


def _gen_noise(x, noise, upper, lower):
    S = x.shape[-1]
    cmax = jnp.max(x, axis=-1, keepdims=True)
    s = jnp.sum(x, axis=-1, keepdims=True)
    q = jnp.sum(x * x, axis=-1, keepdims=True)
    mean = s * (1.0 / S)
    var = jnp.maximum((q - S * mean * mean) * (1.0 / (S - 1)), 0.0)
    std = jnp.sqrt(var)
    clone = jnp.where(x < -cmax * (1.0 / lower), 0.0, x)
    clone = jnp.where(clone > cmax * (1.0 / upper), 0.0, clone)
    return x + clone * (noise * std)


def _conv3x3(a, wmat, bias, H, W):
    patches = _gather3x3(a.astype(jnp.bfloat16), H, W)
    y = jnp.dot(wmat, patches, preferred_element_type=jnp.float32)
    return y + bias


def _row_shift(x, d):
    if d % x.shape[0] == 0:
        return x
    return jnp.concatenate([x[d:], x[:d]], axis=0)


def _bn_cols_folded(y, g, b, cout, eps):
    m = y.shape[0]
    s = jnp.sum(y, axis=0, keepdims=True)
    q = jnp.sum(y * y, axis=0, keepdims=True)
    sc = s[:, 0:cout] + s[:, cout:2 * cout] + s[:, 2 * cout:3 * cout] \
        + s[:, 3 * cout:4 * cout]
    qc = q[:, 0:cout] + q[:, cout:2 * cout] + q[:, 2 * cout:3 * cout] \
        + q[:, 3 * cout:4 * cout]
    cnt = 4.0 * m
    mean = sc * (1.0 / cnt)
    var = jnp.maximum(qc * (1.0 / cnt) - mean * mean, 0.0)
    scale = g * jax.lax.rsqrt(var + eps)
    shift = b - mean * scale
    scale4 = jnp.concatenate([scale] * 4, axis=1)
    shift4 = jnp.concatenate([shift] * 4, axis=1)
    return jnp.maximum(y * scale4 + shift4, 0.0)


def _head_kernel(a_ref, w1_ref, g1_ref, b1_ref, w2_ref, g2_ref, b2_ref,
                 w3_ref, g3_ref, b3_ref, o_ref, *, eps):
    y = jnp.dot(a_ref[...], w1_ref[...], preferred_element_type=jnp.float32)
    m = y.shape[0]
    mean = jnp.sum(y, axis=0, keepdims=True) * (1.0 / m)
    var = jnp.maximum(jnp.sum(y * y, axis=0, keepdims=True) * (1.0 / m)
                      - mean * mean, 0.0)
    scale = g1_ref[...] * jax.lax.rsqrt(var + eps)
    shift = b1_ref[...] - mean * scale
    h1 = jnp.maximum(y * scale + shift, 0.0).astype(jnp.bfloat16)

    row = jax.lax.broadcasted_iota(jnp.int32, (m, 1), 0)
    yy = (row % 16) // 4
    xx = row % 4
    parts = []
    for dy in (-1, 0, 1):
        for dx in (-1, 0, 1):
            shifted = _row_shift(h1, dy * 4 + dx)
            valid = ((yy + dy >= 0) & (yy + dy < 4) &
                     (xx + dx >= 0) & (xx + dx < 4))
            parts.append(jnp.where(valid, shifted, 0.0))
    patches2 = jnp.concatenate(parts, axis=1)
    y2 = jnp.dot(patches2, w2_ref[...], preferred_element_type=jnp.float32)
    h2 = _bn_cols_folded(y2, g2_ref[...], b2_ref[...], 128, eps) \
        .astype(jnp.bfloat16)

    aa = (row % 16) // 4
    bb = row % 4
    y3s = {}
    s_acc = None
    q_acc = None
    for ry in range(2):
        for rx in range(2):
            p3 = []
            for dy in (-1, 0, 1):
                for dx in (-1, 0, 1):
                    ty, tx = ry + dy, rx + dx
                    qy, qx = ty % 2, tx % 2
                    sy, sx = (ty - qy) // 2, (tx - qx) // 2
                    g = (2 * qy + qx) * 128
                    blk = _row_shift(h2[:, g:g + 128], sy * 4 + sx)
                    valid = ((aa + sy >= 0) & (aa + sy < 4) &
                             (bb + sx >= 0) & (bb + sx < 4))
                    p3.append(jnp.where(valid, blk, 0.0))
            patches3 = jnp.concatenate(p3, axis=1)
            y3 = jnp.dot(patches3, w3_ref[...],
                         preferred_element_type=jnp.float32)
            y3s[(ry, rx)] = y3
            s = jnp.sum(y3, axis=0, keepdims=True)
            q = jnp.sum(y3 * y3, axis=0, keepdims=True)
            s_acc = s if s_acc is None else s_acc + s
            q_acc = q if q_acc is None else q_acc + q
    cout = 64
    sc = s_acc[:, 0:cout] + s_acc[:, cout:2 * cout] \
        + s_acc[:, 2 * cout:3 * cout] + s_acc[:, 3 * cout:4 * cout]
    qc = q_acc[:, 0:cout] + q_acc[:, cout:2 * cout] \
        + q_acc[:, 2 * cout:3 * cout] + q_acc[:, 3 * cout:4 * cout]
    cnt = 16.0 * m
    mean = sc * (1.0 / cnt)
    var = jnp.maximum(qc * (1.0 / cnt) - mean * mean, 0.0)
    scale = g3_ref[...] * jax.lax.rsqrt(var + eps)
    shift = b3_ref[...] - mean * scale
    scale4 = jnp.concatenate([scale] * 4, axis=1)
    shift4 = jnp.concatenate([shift] * 4, axis=1)
    for p, (ry, rx) in enumerate([(0, 0), (0, 1), (1, 0), (1, 1)]):
        o_ref[p] = jnp.maximum(y3s[(ry, rx)] * scale4 + shift4,
                               0.0).astype(o_ref.dtype)


def _planar_ct_bn_kernel(x_ref, w_ref, g_ref, b_ref, o_ref, *, H, W,
                         phase_major, eps):
    n_batch = x_ref.shape[0]
    S = H * W
    w = w_ref[...]
    ys, s_acc, q_acc = [], None, None
    for n in range(n_batch):
        patches = _gather3x3(x_ref[n], H, W)
        y = jnp.dot(w, patches, preferred_element_type=jnp.float32)
        ys.append(y)
        s_n = jnp.sum(y, axis=1, keepdims=True)
        q_n = jnp.sum(y * y, axis=1, keepdims=True)
        s_acc = s_n if s_acc is None else s_acc + s_n
        q_acc = q_n if q_acc is None else q_acc + q_n
    r = s_acc.shape[0]
    ri = jax.lax.broadcasted_iota(jnp.int32, (r, r), 0)
    cj = jax.lax.broadcasted_iota(jnp.int32, (r, r), 1)
    if phase_major:
        fold = ((ri % (r // 4)) == (cj % (r // 4))).astype(jnp.float32)
    else:
        fold = ((ri // 4) == (cj // 4)).astype(jnp.float32)
    stats = jnp.dot(fold, jnp.concatenate([s_acc, q_acc], axis=1),
                    preferred_element_type=jnp.float32)
    cnt = float(n_batch * 4 * S)
    mean = stats[:, 0:1] * (1.0 / cnt)
    var = jnp.maximum(stats[:, 1:2] * (1.0 / cnt) - mean * mean, 0.0)
    scale = g_ref[...] * jax.lax.rsqrt(var + eps)
    shift = b_ref[...] - mean * scale
    for n in range(n_batch):
        o_ref[n] = jnp.maximum(ys[n] * scale + shift, 0.0).astype(o_ref.dtype)


def _subphase_ct_bn_kernel(x_ref, w_ref, g_ref, b_ref, o_ref, *, H, W,
                           phase_major, eps):
    n_batch = x_ref.shape[0]
    C = x_ref.shape[1] // 4
    S = H * W
    w = w_ref[...]
    col = jax.lax.broadcasted_iota(jnp.int32, (1, S), 1)
    aa = col // W
    bb = col % W
    ys, s_acc, q_acc = [], None, None
    for n in range(n_batch):
        x = x_ref[n]
        piece_patches = []
        for py in range(2):
            for px in range(2):
                parts = []
                for dy in (-1, 0, 1):
                    for dx in (-1, 0, 1):
                        ty, tx = py + dy, px + dx
                        qy, qx = ty % 2, tx % 2
                        sy, sx = (ty - qy) // 2, (tx - qx) // 2
                        blk = x[(2 * qy + qx) * C:(2 * qy + qx + 1) * C]
                        shifted = _lane_shift(blk, sy * W + sx)
                        valid = ((aa + sy >= 0) & (aa + sy < H) &
                                 (bb + sx >= 0) & (bb + sx < W))
                        parts.append(jnp.where(valid, shifted, 0.0))
                piece_patches.append(jnp.concatenate(parts, axis=0))
        patches = jnp.concatenate(piece_patches, axis=1)
        y = jnp.dot(w, patches, preferred_element_type=jnp.float32)
        ys.append(y)
        s_p = jnp.sum(y, axis=1, keepdims=True)
        q_p = jnp.sum(y * y, axis=1, keepdims=True)
        s_acc = s_p if s_acc is None else s_acc + s_p
        q_acc = q_p if q_acc is None else q_acc + q_p
    r = s_acc.shape[0]
    ri = jax.lax.broadcasted_iota(jnp.int32, (r, r), 0)
    cj = jax.lax.broadcasted_iota(jnp.int32, (r, r), 1)
    if phase_major:
        fold = ((ri % (r // 4)) == (cj % (r // 4))).astype(jnp.float32)
    else:
        fold = ((ri // 4) == (cj // 4)).astype(jnp.float32)
    stats = jnp.dot(fold, jnp.concatenate([s_acc, q_acc], axis=1),
                    preferred_element_type=jnp.float32)
    cnt = float(n_batch * 16 * S)
    mean = stats[:, 0:1] * (1.0 / cnt)
    var = jnp.maximum(stats[:, 1:2] * (1.0 / cnt) - mean * mean, 0.0)
    scale = g_ref[...] * jax.lax.rsqrt(var + eps)
    shift = b_ref[...] - mean * scale
    for n in range(n_batch):
        yb = jnp.maximum(ys[n] * scale + shift, 0.0).astype(o_ref.dtype)
        for p in range(4):
            o_ref[n, p] = yb[:, p * S:(p + 1) * S]


def _ct6_kernel(x_ref, w_ref, o_ref, *, H, W):
    S = H * W
    col = jax.lax.broadcasted_iota(jnp.int32, (1, S), 1)
    aa = col // W
    bb = col % W
    C = x_ref.shape[2] // 4
    w = w_ref[...]
    piece_patches = []
    for py in range(2):
        for px in range(2):
            for py2 in range(2):
                for px2 in range(2):
                    parts = []
                    for dy in (-1, 0, 1):
                        for dx in (-1, 0, 1):
                            ty = 2 * py + py2 + dy
                            tx = 2 * px + px2 + dx
                            qy, qx = ty % 4, tx % 4
                            sy, sx = (ty - qy) // 4, (tx - qx) // 4
                            blk = x_ref[0, 2 * (qy // 2) + (qx // 2),
                                        (2 * (qy % 2) + (qx % 2)) * C:
                                        (2 * (qy % 2) + (qx % 2) + 1) * C]
                            shifted = _lane_shift(blk, sy * W + sx)
                            valid = ((aa + sy >= 0) & (aa + sy < H) &
                                     (bb + sx >= 0) & (bb + sx < W))
                            parts.append(jnp.where(valid, shifted, 0.0))
                    piece_patches.append(jnp.concatenate(parts, axis=0))
    patches = jnp.concatenate(piece_patches, axis=1)
    y = jnp.dot(w, patches, preferred_element_type=jnp.float32)
    yb = y.astype(o_ref.dtype)
    idx = 0
    for py in range(2):
        for px in range(2):
            for py2 in range(2):
                for px2 in range(2):
                    o_ref[0, 2 * py + px, 2 * py2 + px2] = \
                        yb[:, idx * S:(idx + 1) * S]
                    idx += 1


def _tail_kernel(x_ref, n1_ref, n2_ref, w1_ref, b1_ref, w2_ref, b2_ref,
                 w3_ref, b3_ref, w4_ref, b4_ref, o_ref, *, H, W, nc,
                 upper, lower):
    S = H * W
    c2 = n2_ref.shape[1]
    a = x_ref[0].astype(jnp.float32)
    n1 = n1_ref[0].astype(jnp.float32)
    a = _gen_noise(a, n1, upper, lower)
    a = _conv3x3(a, w1_ref[...], b1_ref[...], H, W)
    a = _conv3x3(a, w2_ref[...], b2_ref[...], H, W)
    n2 = jnp.concatenate(
        [n2_ref[0].astype(jnp.float32),
         jnp.zeros((a.shape[0] - c2, S), jnp.float32)], axis=0)
    a = _gen_noise(a, n2, upper, lower)
    a = _conv3x3(a, w3_ref[...], b3_ref[...], H, W)
    a = _conv3x3(a, w4_ref[...], b4_ref[...], H, W)
    o_ref[0] = jnp.tanh(a[:nc, :])


def _head(a1, w1, g1, b1, w2, g2, b2, w3, g3, b3):
    M = a1.shape[0]

    def rep(arr):
        return pl.BlockSpec(arr.shape, lambda i, nd=arr.ndim: (0,) * nd)

    return pl.pallas_call(
        functools.partial(_head_kernel, eps=_EPS),
        out_shape=jax.ShapeDtypeStruct((4, M, 256), jnp.bfloat16),
        grid=(1,),
        in_specs=[rep(a1), rep(w1), rep(g1), rep(b1),
                  rep(w2), rep(g2), rep(b2), rep(w3), rep(g3), rep(b3)],
        out_specs=pl.BlockSpec((4, M, 256), lambda i: (0, 0, 0)),
        compiler_params=pltpu.CompilerParams(
            dimension_semantics=("arbitrary",),
            vmem_limit_bytes=_VMEM_LIMIT),
    )(a1.astype(jnp.bfloat16), w1, g1, b1, w2, g2, b2, w3, g3, b3)


def _planar_ct_bn_relu(x, wpl, gamma_rows, beta_rows, *, H, W,
                       phase_major=False):
    N, Cin, S = x.shape
    R, K = wpl.shape
    return pl.pallas_call(
        functools.partial(_planar_ct_bn_kernel, H=H, W=W,
                          phase_major=phase_major, eps=_EPS),
        out_shape=jax.ShapeDtypeStruct((N, R, S), jnp.bfloat16),
        grid=(1,),
        in_specs=[pl.BlockSpec((N, Cin, S), lambda i: (0, 0, 0)),
                  pl.BlockSpec((R, K), lambda i: (0, 0)),
                  pl.BlockSpec((R, 1), lambda i: (0, 0)),
                  pl.BlockSpec((R, 1), lambda i: (0, 0))],
        out_specs=pl.BlockSpec((N, R, S), lambda i: (0, 0, 0)),
        compiler_params=pltpu.CompilerParams(
            dimension_semantics=("arbitrary",),
            vmem_limit_bytes=_VMEM_LIMIT),
    )(x, wpl, gamma_rows, beta_rows)


def _subphase_ct_bn_relu(x, wpl, gamma_rows, beta_rows, *, H, W,
                         phase_major=False):
    N, C4, S = x.shape
    R, K = wpl.shape
    return pl.pallas_call(
        functools.partial(_subphase_ct_bn_kernel, H=H, W=W,
                          phase_major=phase_major, eps=_EPS),
        out_shape=jax.ShapeDtypeStruct((N, 4, R, S), jnp.bfloat16),
        grid=(1,),
        in_specs=[pl.BlockSpec((N, C4, S), lambda i: (0, 0, 0)),
                  pl.BlockSpec((R, K), lambda i: (0, 0)),
                  pl.BlockSpec((R, 1), lambda i: (0, 0)),
                  pl.BlockSpec((R, 1), lambda i: (0, 0))],
        out_specs=pl.BlockSpec((N, 4, R, S), lambda i: (0, 0, 0, 0)),
        compiler_params=pltpu.CompilerParams(
            dimension_semantics=("arbitrary",),
            vmem_limit_bytes=_VMEM_LIMIT),
    )(x, wpl, gamma_rows, beta_rows)


def _ct6_phase(x, wpl, *, H, W):
    N, P4, R5, S = x.shape
    R, K = wpl.shape
    return pl.pallas_call(
        functools.partial(_ct6_kernel, H=H, W=W),
        out_shape=jax.ShapeDtypeStruct((N, 4, 4, R, S), jnp.bfloat16),
        grid=(N,),
        in_specs=[pl.BlockSpec((1, P4, R5, S), lambda n: (n, 0, 0, 0)),
                  pl.BlockSpec((R, K), lambda n: (0, 0))],
        out_specs=pl.BlockSpec((1, 4, 4, R, S), lambda n: (n, 0, 0, 0, 0)),
        compiler_params=pltpu.CompilerParams(
            dimension_semantics=("parallel",),
            vmem_limit_bytes=_VMEM_LIMIT),
    )(x, wpl)


def _tail(act, n1, n2, weights, *, nc, H, W, upper=4.0, lower=2.0):
    N, C0, S = act.shape
    c2 = n2.shape[1]
    w1, b1, w2, b2, w3, b3, w4, b4 = weights

    def rep_spec(arr):
        nd = arr.ndim
        return pl.BlockSpec(arr.shape, lambda n, nd=nd: (0,) * nd)

    return pl.pallas_call(
        functools.partial(_tail_kernel, H=H, W=W, nc=nc,
                          upper=upper, lower=lower),
        out_shape=jax.ShapeDtypeStruct((N, nc, S), jnp.float32),
        grid=(N,),
        in_specs=[pl.BlockSpec((1, C0, S), lambda n: (n, 0, 0)),
                  pl.BlockSpec((1, C0, S), lambda n: (n, 0, 0)),
                  pl.BlockSpec((1, c2, S), lambda n: (n, 0, 0)),
                  rep_spec(w1), rep_spec(b1), rep_spec(w2), rep_spec(b2),
                  rep_spec(w3), rep_spec(b3), rep_spec(w4), rep_spec(b4)],
        out_specs=pl.BlockSpec((1, nc, S), lambda n: (n, 0, 0)),
        compiler_params=pltpu.CompilerParams(
            dimension_semantics=("parallel",),
            vmem_limit_bytes=_VMEM_LIMIT),
    )(act, n1, n2, w1, b1, w2, b2, w3, b3, w4, b4)


def kernel(m1, m2, m3, m4, m5, m6,
           g1, b1, g2, b2, g3, b3, g4, b4, g5, b5,
           c1_w, c1_b, c2_w, c2_b, c3_w, c3_b, c4_w, c4_b,
           x, noise1, noise2):
    nc, ngf = 1, 16
    N, nz = x.shape[0], x.shape[1]
    z = x.reshape(N, nz).astype(jnp.bfloat16)

    eye16 = jnp.eye(16, dtype=z.dtype)
    a1 = (eye16[None, :, :, None] * z[:, None, None, :]).reshape(
        N * 16, 16 * nz)
    y3 = _head(a1, m1, g1, b1, m2, g2, b2, m3, g3, b3)
    hp = (y3.reshape(2, 2, N, 4, 4, 2, 2, 64)
          .transpose(2, 7, 3, 0, 5, 4, 1, 6)
          .reshape(N, ngf * 4, 256))

    m4pm = m4.reshape(ngf * 2, 4, m4.shape[1]).transpose(1, 0, 2) \
        .reshape(ngf * 8, m4.shape[1])
    g4pm = g4.reshape(ngf * 2, 4).transpose(1, 0).reshape(ngf * 8, 1)
    b4pm = b4.reshape(ngf * 2, 4).transpose(1, 0).reshape(ngf * 8, 1)
    y = _planar_ct_bn_relu(hp, m4pm, g4pm, b4pm, H=16, W=16,
                           phase_major=True)
    m5pm = m5.reshape(ngf, 4, m5.shape[1]).transpose(1, 0, 2) \
        .reshape(ngf * 4, m5.shape[1])
    g5pm = g5.reshape(ngf, 4).transpose(1, 0).reshape(ngf * 4, 1)
    b5pm = b5.reshape(ngf, 4).transpose(1, 0).reshape(ngf * 4, 1)
    y5p = _subphase_ct_bn_relu(y, m5pm, g5pm, b5pm, H=16, W=16,
                               phase_major=True)

    y6 = _ct6_phase(y5p, m6, H=16, W=16)
    act = (y6.reshape(N, 2, 2, 2, 2, 8, 2, 2, 16, 16)
           .transpose(0, 5, 8, 1, 3, 6, 9, 2, 4, 7)
           .reshape(N, ngf // 2, 128 * 128))

    S = 128 * 128
    n1 = noise1.reshape(N, ngf // 2, S)
    n2 = noise2.reshape(N, noise2.shape[1], S)
    weights = (c1_w, c1_b, c2_w, c2_b, c3_w, c3_b, c4_w, c4_b)
    out = _tail(act, n1, n2, weights, nc=nc, H=128, W=128)
    return out.reshape(N, nc, 128, 128)


# speedup vs baseline: 1.1101x; 1.0282x over previous
import functools

import jax
import jax.numpy as jnp
from jax.experimental import pallas as pl
from jax.experimental.pallas import tpu as pltpu

_EPS = 1e-5
_VMEM_LIMIT = 48 * 1024 * 1024


def _lane_shift(x, d):
    S = x.shape[-1]
    d = d % S
    if d == 0:
        return x
    return jnp.concatenate([x[:, d:], x[:, :d]], axis=-1)


def _gather3x3(x, H, W):
    S = H * W
    col = jax.lax.broadcasted_iota(jnp.int32, (1, S), 1)
    yy = col // W
    xx = col % W
    parts = []
    for wy in range(3):
        for wx in range(3):
            dy, dx = wy - 1, wx - 1
            shifted = _lane_shift(x, dy * W + dx)
            valid = ((yy + dy >= 0) & (yy + dy < H) &
                     (xx + dx >= 0) & (xx + dx < W))
            parts.append(jnp.where(valid, shifted, 0.0))
    return jnp.concatenate(parts, axis=0)


def _gen_noise(x, noise, upper, lower):
    S = x.shape[-1]
    cmax = jnp.max(x, axis=-1, keepdims=True)
    s = jnp.sum(x, axis=-1, keepdims=True)
    q = jnp.sum(x * x, axis=-1, keepdims=True)
    mean = s * (1.0 / S)
    var = jnp.maximum((q - S * mean * mean) * (1.0 / (S - 1)), 0.0)
    std = jnp.sqrt(var)
    clone = jnp.where(x < -cmax * (1.0 / lower), 0.0, x)
    clone = jnp.where(clone > cmax * (1.0 / upper), 0.0, clone)
    return x + clone * (noise * std)


def _conv3x3(a, wmat, bias, H, W):
    patches = _gather3x3(a.astype(jnp.bfloat16), H, W)
    y = jnp.dot(wmat, patches, preferred_element_type=jnp.float32)
    return y + bias


def _row_shift(x, d):
    if d % x.shape[0] == 0:
        return x
    return jnp.concatenate([x[d:], x[:d]], axis=0)


def _bn_cols_folded(y, g, b, cout, eps):
    m = y.shape[0]
    s = jnp.sum(y, axis=0, keepdims=True)
    q = jnp.sum(y * y, axis=0, keepdims=True)
    sc = s[:, 0:cout] + s[:, cout:2 * cout] + s[:, 2 * cout:3 * cout] \
        + s[:, 3 * cout:4 * cout]
    qc = q[:, 0:cout] + q[:, cout:2 * cout] + q[:, 2 * cout:3 * cout] \
        + q[:, 3 * cout:4 * cout]
    cnt = 4.0 * m
    mean = sc * (1.0 / cnt)
    var = jnp.maximum(qc * (1.0 / cnt) - mean * mean, 0.0)
    scale = g * jax.lax.rsqrt(var + eps)
    shift = b - mean * scale
    scale4 = jnp.concatenate([scale] * 4, axis=1)
    shift4 = jnp.concatenate([shift] * 4, axis=1)
    return jnp.maximum(y * scale4 + shift4, 0.0)


def _head_kernel(a_ref, w1_ref, g1_ref, b1_ref, w2_ref, g2_ref, b2_ref,
                 w3_ref, g3_ref, b3_ref, o_ref, *, eps):
    y = jnp.dot(a_ref[...], w1_ref[...], preferred_element_type=jnp.float32)
    m = y.shape[0]
    mean = jnp.sum(y, axis=0, keepdims=True) * (1.0 / m)
    var = jnp.maximum(jnp.sum(y * y, axis=0, keepdims=True) * (1.0 / m)
                      - mean * mean, 0.0)
    scale = g1_ref[...] * jax.lax.rsqrt(var + eps)
    shift = b1_ref[...] - mean * scale
    h1 = jnp.maximum(y * scale + shift, 0.0).astype(jnp.bfloat16)

    row = jax.lax.broadcasted_iota(jnp.int32, (m, 1), 0)
    yy = (row % 16) // 4
    xx = row % 4
    parts = []
    for dy in (-1, 0, 1):
        for dx in (-1, 0, 1):
            shifted = _row_shift(h1, dy * 4 + dx)
            valid = ((yy + dy >= 0) & (yy + dy < 4) &
                     (xx + dx >= 0) & (xx + dx < 4))
            parts.append(jnp.where(valid, shifted, 0.0))
    patches2 = jnp.concatenate(parts, axis=1)
    y2 = jnp.dot(patches2, w2_ref[...], preferred_element_type=jnp.float32)
    h2 = _bn_cols_folded(y2, g2_ref[...], b2_ref[...], 128, eps) \
        .astype(jnp.bfloat16)

    aa = (row % 16) // 4
    bb = row % 4
    y3s = {}
    s_acc = None
    q_acc = None
    for ry in range(2):
        for rx in range(2):
            p3 = []
            for dy in (-1, 0, 1):
                for dx in (-1, 0, 1):
                    ty, tx = ry + dy, rx + dx
                    qy, qx = ty % 2, tx % 2
                    sy, sx = (ty - qy) // 2, (tx - qx) // 2
                    g = (2 * qy + qx) * 128
                    blk = _row_shift(h2[:, g:g + 128], sy * 4 + sx)
                    valid = ((aa + sy >= 0) & (aa + sy < 4) &
                             (bb + sx >= 0) & (bb + sx < 4))
                    p3.append(jnp.where(valid, blk, 0.0))
            patches3 = jnp.concatenate(p3, axis=1)
            y3 = jnp.dot(patches3, w3_ref[...],
                         preferred_element_type=jnp.float32)
            y3s[(ry, rx)] = y3
            s = jnp.sum(y3, axis=0, keepdims=True)
            q = jnp.sum(y3 * y3, axis=0, keepdims=True)
            s_acc = s if s_acc is None else s_acc + s
            q_acc = q if q_acc is None else q_acc + q
    cout = 64
    sc = s_acc[:, 0:cout] + s_acc[:, cout:2 * cout] \
        + s_acc[:, 2 * cout:3 * cout] + s_acc[:, 3 * cout:4 * cout]
    qc = q_acc[:, 0:cout] + q_acc[:, cout:2 * cout] \
        + q_acc[:, 2 * cout:3 * cout] + q_acc[:, 3 * cout:4 * cout]
    cnt = 16.0 * m
    mean = sc * (1.0 / cnt)
    var = jnp.maximum(qc * (1.0 / cnt) - mean * mean, 0.0)
    scale = g3_ref[...] * jax.lax.rsqrt(var + eps)
    shift = b3_ref[...] - mean * scale
    scale4 = jnp.concatenate([scale] * 4, axis=1)
    shift4 = jnp.concatenate([shift] * 4, axis=1)
    for p, (ry, rx) in enumerate([(0, 0), (0, 1), (1, 0), (1, 1)]):
        o_ref[p] = jnp.maximum(y3s[(ry, rx)] * scale4 + shift4,
                               0.0).astype(o_ref.dtype)


def _mid_kernel(x_ref, w4_ref, g4_ref, b4_ref, w_ref, g_ref, b_ref, o_ref,
                *, H, W, eps):
    n_batch = x_ref.shape[0]
    S = H * W
    w4 = w4_ref[...]
    y4s, s4_acc, q4_acc = [], None, None
    for n in range(n_batch):
        patches = _gather3x3(x_ref[n], H, W)
        y4 = jnp.dot(w4, patches, preferred_element_type=jnp.float32)
        y4s.append(y4)
        s_n = jnp.sum(y4, axis=1, keepdims=True)
        q_n = jnp.sum(y4 * y4, axis=1, keepdims=True)
        s4_acc = s_n if s4_acc is None else s4_acc + s_n
        q4_acc = q_n if q4_acc is None else q4_acc + q_n
    r4 = s4_acc.shape[0]
    ri = jax.lax.broadcasted_iota(jnp.int32, (r4, r4), 0)
    cj = jax.lax.broadcasted_iota(jnp.int32, (r4, r4), 1)
    fold4 = ((ri % (r4 // 4)) == (cj % (r4 // 4))).astype(jnp.float32)
    st4 = jnp.dot(fold4, jnp.concatenate([s4_acc, q4_acc], axis=1),
                  preferred_element_type=jnp.float32)
    cnt4 = float(n_batch * 4 * S)
    mean4 = st4[:, 0:1] * (1.0 / cnt4)
    var4 = jnp.maximum(st4[:, 1:2] * (1.0 / cnt4) - mean4 * mean4, 0.0)
    scale4 = g4_ref[...] * jax.lax.rsqrt(var4 + eps)
    shift4 = b4_ref[...] - mean4 * scale4

    C = r4 // 4
    w = w_ref[...]
    col = jax.lax.broadcasted_iota(jnp.int32, (1, S), 1)
    aa = col // W
    bb = col % W
    ys, s_acc, q_acc = [], None, None
    for n in range(n_batch):
        x = jnp.maximum(y4s[n] * scale4 + shift4, 0.0).astype(jnp.bfloat16)
        piece_patches = []
        for py in range(2):
            for px in range(2):
                parts = []
                for dy in (-1, 0, 1):
                    for dx in (-1, 0, 1):
                        ty, tx = py + dy, px + dx
                        qy, qx = ty % 2, tx % 2
                        sy, sx = (ty - qy) // 2, (tx - qx) // 2
                        blk = x[(2 * qy + qx) * C:(2 * qy + qx + 1) * C]
                        shifted = _lane_shift(blk, sy * W + sx)
                        valid = ((aa + sy >= 0) & (aa + sy < H) &
                                 (bb + sx >= 0) & (bb + sx < W))
                        parts.append(jnp.where(valid, shifted, 0.0))
                piece_patches.append(jnp.concatenate(parts, axis=0))
        patches = jnp.concatenate(piece_patches, axis=1)
        y = jnp.dot(w, patches, preferred_element_type=jnp.float32)
        ys.append(y)
        s_p = jnp.sum(y, axis=1, keepdims=True)
        q_p = jnp.sum(y * y, axis=1, keepdims=True)
        s_acc = s_p if s_acc is None else s_acc + s_p
        q_acc = q_p if q_acc is None else q_acc + q_p
    r = s_acc.shape[0]
    ri = jax.lax.broadcasted_iota(jnp.int32, (r, r), 0)
    cj = jax.lax.broadcasted_iota(jnp.int32, (r, r), 1)
    fold = ((ri % (r // 4)) == (cj % (r // 4))).astype(jnp.float32)
    stats = jnp.dot(fold, jnp.concatenate([s_acc, q_acc], axis=1),
                    preferred_element_type=jnp.float32)
    cnt = float(n_batch * 16 * S)
    mean = stats[:, 0:1] * (1.0 / cnt)
    var = jnp.maximum(stats[:, 1:2] * (1.0 / cnt) - mean * mean, 0.0)
    scale = g_ref[...] * jax.lax.rsqrt(var + eps)
    shift = b_ref[...] - mean * scale
    for n in range(n_batch):
        yb = jnp.maximum(ys[n] * scale + shift, 0.0).astype(o_ref.dtype)
        for p in range(4):
            o_ref[n, p] = yb[:, p * S:(p + 1) * S]


def _ct6_kernel(x_ref, w_ref, o_ref, *, H, W):
    S = H * W
    col = jax.lax.broadcasted_iota(jnp.int32, (1, S), 1)
    aa = col // W
    bb = col % W
    C = x_ref.shape[2] // 4
    w = w_ref[...]
    piece_patches = []
    for py in range(2):
        for px in range(2):
            for py2 in range(2):
                for px2 in range(2):
                    parts = []
                    for dy in (-1, 0, 1):
                        for dx in (-1, 0, 1):
                            ty = 2 * py + py2 + dy
                            tx = 2 * px + px2 + dx
                            qy, qx = ty % 4, tx % 4
                            sy, sx = (ty - qy) // 4, (tx - qx) // 4
                            blk = x_ref[0, 2 * (qy // 2) + (qx // 2),
                                        (2 * (qy % 2) + (qx % 2)) * C:
                                        (2 * (qy % 2) + (qx % 2) + 1) * C]
                            shifted = _lane_shift(blk, sy * W + sx)
                            valid = ((aa + sy >= 0) & (aa + sy < H) &
                                     (bb + sx >= 0) & (bb + sx < W))
                            parts.append(jnp.where(valid, shifted, 0.0))
                    piece_patches.append(jnp.concatenate(parts, axis=0))
    patches = jnp.concatenate(piece_patches, axis=1)
    y = jnp.dot(w, patches, preferred_element_type=jnp.float32)
    yb = y.astype(o_ref.dtype)
    idx = 0
    for py in range(2):
        for px in range(2):
            for py2 in range(2):
                for px2 in range(2):
                    o_ref[0, 2 * py + px, 2 * py2 + px2] = \
                        yb[:, idx * S:(idx + 1) * S]
                    idx += 1


def _tail_kernel(x_ref, n1_ref, n2_ref, w1_ref, b1_ref, w2_ref, b2_ref,
                 w3_ref, b3_ref, w4_ref, b4_ref, o_ref, *, H, W, nc,
                 upper, lower):
    S = H * W
    c2 = n2_ref.shape[1]
    a = x_ref[0].astype(jnp.float32)
    n1 = n1_ref[0].astype(jnp.float32)
    a = _gen_noise(a, n1, upper, lower)
    a = _conv3x3(a, w1_ref[...], b1_ref[...], H, W)
    a = _conv3x3(a, w2_ref[...], b2_ref[...], H, W)
    n2 = jnp.concatenate(
        [n2_ref[0].astype(jnp.float32),
         jnp.zeros((a.shape[0] - c2, S), jnp.float32)], axis=0)
    a = _gen_noise(a, n2, upper, lower)
    a = _conv3x3(a, w3_ref[...], b3_ref[...], H, W)
    a = _conv3x3(a, w4_ref[...], b4_ref[...], H, W)
    o_ref[0] = jnp.tanh(a[:nc, :])


def _head(a1, w1, g1, b1, w2, g2, b2, w3, g3, b3):
    M = a1.shape[0]

    def rep(arr):
        return pl.BlockSpec(arr.shape, lambda i, nd=arr.ndim: (0,) * nd)

    return pl.pallas_call(
        functools.partial(_head_kernel, eps=_EPS),
        out_shape=jax.ShapeDtypeStruct((4, M, 256), jnp.bfloat16),
        grid=(1,),
        in_specs=[rep(a1), rep(w1), rep(g1), rep(b1),
                  rep(w2), rep(g2), rep(b2), rep(w3), rep(g3), rep(b3)],
        out_specs=pl.BlockSpec((4, M, 256), lambda i: (0, 0, 0)),
        compiler_params=pltpu.CompilerParams(
            dimension_semantics=("arbitrary",),
            vmem_limit_bytes=_VMEM_LIMIT),
    )(a1.astype(jnp.bfloat16), w1, g1, b1, w2, g2, b2, w3, g3, b3)


def _mid(x, w4, g4r, b4r, wpl, gamma_rows, beta_rows, *, H, W):
    N, Cin, S = x.shape
    R4, K4 = w4.shape
    R, K = wpl.shape
    return pl.pallas_call(
        functools.partial(_mid_kernel, H=H, W=W, eps=_EPS),
        out_shape=jax.ShapeDtypeStruct((N, 4, R, S), jnp.bfloat16),
        grid=(1,),
        in_specs=[pl.BlockSpec((N, Cin, S), lambda i: (0, 0, 0)),
                  pl.BlockSpec((R4, K4), lambda i: (0, 0)),
                  pl.BlockSpec((R4, 1), lambda i: (0, 0)),
                  pl.BlockSpec((R4, 1), lambda i: (0, 0)),
                  pl.BlockSpec((R, K), lambda i: (0, 0)),
                  pl.BlockSpec((R, 1), lambda i: (0, 0)),
                  pl.BlockSpec((R, 1), lambda i: (0, 0))],
        out_specs=pl.BlockSpec((N, 4, R, S), lambda i: (0, 0, 0, 0)),
        compiler_params=pltpu.CompilerParams(
            dimension_semantics=("arbitrary",),
            vmem_limit_bytes=_VMEM_LIMIT),
    )(x, w4, g4r, b4r, wpl, gamma_rows, beta_rows)


def _ct6_phase(x, wpl, *, H, W):
    N, P4, R5, S = x.shape
    R, K = wpl.shape
    return pl.pallas_call(
        functools.partial(_ct6_kernel, H=H, W=W),
        out_shape=jax.ShapeDtypeStruct((N, 4, 4, R, S), jnp.bfloat16),
        grid=(N,),
        in_specs=[pl.BlockSpec((1, P4, R5, S), lambda n: (n, 0, 0, 0)),
                  pl.BlockSpec((R, K), lambda n: (0, 0))],
        out_specs=pl.BlockSpec((1, 4, 4, R, S), lambda n: (n, 0, 0, 0, 0)),
        compiler_params=pltpu.CompilerParams(
            dimension_semantics=("parallel",),
            vmem_limit_bytes=_VMEM_LIMIT),
    )(x, wpl)


def _tail(act, n1, n2, weights, *, nc, H, W, upper=4.0, lower=2.0):
    N, C0, S = act.shape
    c2 = n2.shape[1]
    w1, b1, w2, b2, w3, b3, w4, b4 = weights

    def rep_spec(arr):
        nd = arr.ndim
        return pl.BlockSpec(arr.shape, lambda n, nd=nd: (0,) * nd)

    return pl.pallas_call(
        functools.partial(_tail_kernel, H=H, W=W, nc=nc,
                          upper=upper, lower=lower),
        out_shape=jax.ShapeDtypeStruct((N, nc, S), jnp.float32),
        grid=(N,),
        in_specs=[pl.BlockSpec((1, C0, S), lambda n: (n, 0, 0)),
                  pl.BlockSpec((1, C0, S), lambda n: (n, 0, 0)),
                  pl.BlockSpec((1, c2, S), lambda n: (n, 0, 0)),
                  rep_spec(w1), rep_spec(b1), rep_spec(w2), rep_spec(b2),
                  rep_spec(w3), rep_spec(b3), rep_spec(w4), rep_spec(b4)],
        out_specs=pl.BlockSpec((1, nc, S), lambda n: (n, 0, 0)),
        compiler_params=pltpu.CompilerParams(
            dimension_semantics=("parallel",),
            vmem_limit_bytes=_VMEM_LIMIT),
    )(act, n1, n2, w1, b1, w2, b2, w3, b3, w4, b4)


def kernel(m1, m2, m3, m4, m5, m6,
           g1, b1, g2, b2, g3, b3, g4, b4, g5, b5,
           c1_w, c1_b, c2_w, c2_b, c3_w, c3_b, c4_w, c4_b,
           x, noise1, noise2):
    nc, ngf = 1, 16
    N, nz = x.shape[0], x.shape[1]
    z = x.reshape(N, nz).astype(jnp.bfloat16)

    eye16 = jnp.eye(16, dtype=z.dtype)
    a1 = (eye16[None, :, :, None] * z[:, None, None, :]).reshape(
        N * 16, 16 * nz)
    y3 = _head(a1, m1, g1, b1, m2, g2, b2, m3, g3, b3)
    hp = (y3.reshape(2, 2, N, 4, 4, 2, 2, 64)
          .transpose(2, 7, 3, 0, 5, 4, 1, 6)
          .reshape(N, ngf * 4, 256))

    m4pm = m4.reshape(ngf * 2, 4, m4.shape[1]).transpose(1, 0, 2) \
        .reshape(ngf * 8, m4.shape[1])
    g4pm = g4.reshape(ngf * 2, 4).transpose(1, 0).reshape(ngf * 8, 1)
    b4pm = b4.reshape(ngf * 2, 4).transpose(1, 0).reshape(ngf * 8, 1)
    m5pm = m5.reshape(ngf, 4, m5.shape[1]).transpose(1, 0, 2) \
        .reshape(ngf * 4, m5.shape[1])
    g5pm = g5.reshape(ngf, 4).transpose(1, 0).reshape(ngf * 4, 1)
    b5pm = b5.reshape(ngf, 4).transpose(1, 0).reshape(ngf * 4, 1)
    y5p = _mid(hp, m4pm, g4pm, b4pm, m5pm, g5pm, b5pm,
               H=16, W=16)

    y6 = _ct6_phase(y5p, m6, H=16, W=16)
    act = (y6.reshape(N, 2, 2, 2, 2, 8, 2, 2, 16, 16)
           .transpose(0, 5, 8, 1, 3, 6, 9, 2, 4, 7)
           .reshape(N, ngf // 2, 128 * 128))

    S = 128 * 128
    n1 = noise1.reshape(N, ngf // 2, S)
    n2 = noise2.reshape(N, noise2.shape[1], S)
    weights = (c1_w, c1_b, c2_w, c2_b, c3_w, c3_b, c4_w, c4_b)
    out = _tail(act, n1, n2, weights, nc=nc, H=128, W=128)
    return out.reshape(N, nc, 128, 128)


# speedup vs baseline: 1.1407x; 1.0275x over previous
import functools

import jax
import jax.numpy as jnp
from jax.experimental import pallas as pl
from jax.experimental.pallas import tpu as pltpu

_EPS = 1e-5
_VMEM_LIMIT = 48 * 1024 * 1024


def _lane_shift(x, d):
    S = x.shape[-1]
    d = d % S
    if d == 0:
        return x
    return jnp.concatenate([x[:, d:], x[:, :d]], axis=-1)


def _gather3x3(x, H, W):
    S = H * W
    col = jax.lax.broadcasted_iota(jnp.int32, (1, S), 1)
    yy = col // W
    xx = col % W
    parts = []
    for wy in range(3):
        for wx in range(3):
            dy, dx = wy - 1, wx - 1
            shifted = _lane_shift(x, dy * W + dx)
            valid = ((yy + dy >= 0) & (yy + dy < H) &
                     (xx + dx >= 0) & (xx + dx < W))
            parts.append(jnp.where(valid, shifted, 0.0))
    return jnp.concatenate(parts, axis=0)


def _gen_noise(x, noise, upper, lower):
    S = x.shape[-1]
    cmax = jnp.max(x, axis=-1, keepdims=True)
    s = jnp.sum(x, axis=-1, keepdims=True)
    q = jnp.sum(x * x, axis=-1, keepdims=True)
    mean = s * (1.0 / S)
    var = jnp.maximum((q - S * mean * mean) * (1.0 / (S - 1)), 0.0)
    std = jnp.sqrt(var)
    clone = jnp.where(x < -cmax * (1.0 / lower), 0.0, x)
    clone = jnp.where(clone > cmax * (1.0 / upper), 0.0, clone)
    return x + clone * (noise * std)


def _conv3x3(a, wmat, bias, H, W):
    patches = _gather3x3(a.astype(jnp.bfloat16), H, W)
    y = jnp.dot(wmat, patches, preferred_element_type=jnp.float32)
    return y + bias


def _row_shift(x, d):
    if d % x.shape[0] == 0:
        return x
    return jnp.concatenate([x[d:], x[:d]], axis=0)


def _bn_cols_folded(y, g, b, cout, eps):
    m = y.shape[0]
    s = jnp.sum(y, axis=0, keepdims=True)
    q = jnp.sum(y * y, axis=0, keepdims=True)
    sc = s[:, 0:cout] + s[:, cout:2 * cout] + s[:, 2 * cout:3 * cout] \
        + s[:, 3 * cout:4 * cout]
    qc = q[:, 0:cout] + q[:, cout:2 * cout] + q[:, 2 * cout:3 * cout] \
        + q[:, 3 * cout:4 * cout]
    cnt = 4.0 * m
    mean = sc * (1.0 / cnt)
    var = jnp.maximum(qc * (1.0 / cnt) - mean * mean, 0.0)
    scale = g * jax.lax.rsqrt(var + eps)
    shift = b - mean * scale
    scale4 = jnp.concatenate([scale] * 4, axis=1)
    shift4 = jnp.concatenate([shift] * 4, axis=1)
    return jnp.maximum(y * scale4 + shift4, 0.0)


def _head_kernel(a_ref, w1_ref, g1_ref, b1_ref, w2_ref, g2_ref, b2_ref,
                 w3_ref, g3_ref, b3_ref, o_ref, *, eps):
    y = jnp.dot(a_ref[...], w1_ref[...], preferred_element_type=jnp.float32)
    m = y.shape[0]
    mean = jnp.sum(y, axis=0, keepdims=True) * (1.0 / m)
    var = jnp.maximum(jnp.sum(y * y, axis=0, keepdims=True) * (1.0 / m)
                      - mean * mean, 0.0)
    scale = g1_ref[...] * jax.lax.rsqrt(var + eps)
    shift = b1_ref[...] - mean * scale
    h1 = jnp.maximum(y * scale + shift, 0.0).astype(jnp.bfloat16)

    row = jax.lax.broadcasted_iota(jnp.int32, (m, 1), 0)
    yy = (row % 16) // 4
    xx = row % 4
    parts = []
    for dy in (-1, 0, 1):
        for dx in (-1, 0, 1):
            shifted = _row_shift(h1, dy * 4 + dx)
            valid = ((yy + dy >= 0) & (yy + dy < 4) &
                     (xx + dx >= 0) & (xx + dx < 4))
            parts.append(jnp.where(valid, shifted, 0.0))
    patches2 = jnp.concatenate(parts, axis=1)
    y2 = jnp.dot(patches2, w2_ref[...], preferred_element_type=jnp.float32)
    h2 = _bn_cols_folded(y2, g2_ref[...], b2_ref[...], 128, eps) \
        .astype(jnp.bfloat16)

    aa = (row % 16) // 4
    bb = row % 4
    y3s = {}
    s_acc = None
    q_acc = None
    for ry in range(2):
        for rx in range(2):
            p3 = []
            for dy in (-1, 0, 1):
                for dx in (-1, 0, 1):
                    ty, tx = ry + dy, rx + dx
                    qy, qx = ty % 2, tx % 2
                    sy, sx = (ty - qy) // 2, (tx - qx) // 2
                    g = (2 * qy + qx) * 128
                    blk = _row_shift(h2[:, g:g + 128], sy * 4 + sx)
                    valid = ((aa + sy >= 0) & (aa + sy < 4) &
                             (bb + sx >= 0) & (bb + sx < 4))
                    p3.append(jnp.where(valid, blk, 0.0))
            patches3 = jnp.concatenate(p3, axis=1)
            y3 = jnp.dot(patches3, w3_ref[...],
                         preferred_element_type=jnp.float32)
            y3s[(ry, rx)] = y3
            s = jnp.sum(y3, axis=0, keepdims=True)
            q = jnp.sum(y3 * y3, axis=0, keepdims=True)
            s_acc = s if s_acc is None else s_acc + s
            q_acc = q if q_acc is None else q_acc + q
    cout = 64
    sc = s_acc[:, 0:cout] + s_acc[:, cout:2 * cout] \
        + s_acc[:, 2 * cout:3 * cout] + s_acc[:, 3 * cout:4 * cout]
    qc = q_acc[:, 0:cout] + q_acc[:, cout:2 * cout] \
        + q_acc[:, 2 * cout:3 * cout] + q_acc[:, 3 * cout:4 * cout]
    cnt = 16.0 * m
    mean = sc * (1.0 / cnt)
    var = jnp.maximum(qc * (1.0 / cnt) - mean * mean, 0.0)
    scale = g3_ref[...] * jax.lax.rsqrt(var + eps)
    shift = b3_ref[...] - mean * scale
    scale4 = jnp.concatenate([scale] * 4, axis=1)
    shift4 = jnp.concatenate([shift] * 4, axis=1)
    for p, (ry, rx) in enumerate([(0, 0), (0, 1), (1, 0), (1, 1)]):
        o_ref[p] = jnp.maximum(y3s[(ry, rx)] * scale4 + shift4,
                               0.0).astype(o_ref.dtype)


def _mid_kernel(x_ref, w4_ref, g4_ref, b4_ref, w_ref, g_ref, b_ref, o_ref,
                *, H, W, eps):
    n_batch = x_ref.shape[0]
    S = H * W
    w4 = w4_ref[...]
    y4s, s4_acc, q4_acc = [], None, None
    for n in range(n_batch):
        patches = _gather3x3(x_ref[n], H, W)
        y4 = jnp.dot(w4, patches, preferred_element_type=jnp.float32)
        y4s.append(y4)
        s_n = jnp.sum(y4, axis=1, keepdims=True)
        q_n = jnp.sum(y4 * y4, axis=1, keepdims=True)
        s4_acc = s_n if s4_acc is None else s4_acc + s_n
        q4_acc = q_n if q4_acc is None else q4_acc + q_n
    r4 = s4_acc.shape[0]
    ri = jax.lax.broadcasted_iota(jnp.int32, (r4, r4), 0)
    cj = jax.lax.broadcasted_iota(jnp.int32, (r4, r4), 1)
    fold4 = ((ri % (r4 // 4)) == (cj % (r4 // 4))).astype(jnp.float32)
    st4 = jnp.dot(fold4, jnp.concatenate([s4_acc, q4_acc], axis=1),
                  preferred_element_type=jnp.float32)
    cnt4 = float(n_batch * 4 * S)
    mean4 = st4[:, 0:1] * (1.0 / cnt4)
    var4 = jnp.maximum(st4[:, 1:2] * (1.0 / cnt4) - mean4 * mean4, 0.0)
    scale4 = g4_ref[...] * jax.lax.rsqrt(var4 + eps)
    shift4 = b4_ref[...] - mean4 * scale4

    C = r4 // 4
    w = w_ref[...]
    col = jax.lax.broadcasted_iota(jnp.int32, (1, S), 1)
    aa = col // W
    bb = col % W
    ys, s_acc, q_acc = [], None, None
    for n in range(n_batch):
        x = jnp.maximum(y4s[n] * scale4 + shift4, 0.0).astype(jnp.bfloat16)
        piece_patches = []
        for py in range(2):
            for px in range(2):
                parts = []
                for dy in (-1, 0, 1):
                    for dx in (-1, 0, 1):
                        ty, tx = py + dy, px + dx
                        qy, qx = ty % 2, tx % 2
                        sy, sx = (ty - qy) // 2, (tx - qx) // 2
                        blk = x[(2 * qy + qx) * C:(2 * qy + qx + 1) * C]
                        shifted = _lane_shift(blk, sy * W + sx)
                        valid = ((aa + sy >= 0) & (aa + sy < H) &
                                 (bb + sx >= 0) & (bb + sx < W))
                        parts.append(jnp.where(valid, shifted, 0.0))
                piece_patches.append(jnp.concatenate(parts, axis=0))
        patches = jnp.concatenate(piece_patches, axis=1)
        y = jnp.dot(w, patches, preferred_element_type=jnp.float32)
        ys.append(y)
        s_p = jnp.sum(y, axis=1, keepdims=True)
        q_p = jnp.sum(y * y, axis=1, keepdims=True)
        s_acc = s_p if s_acc is None else s_acc + s_p
        q_acc = q_p if q_acc is None else q_acc + q_p
    r = s_acc.shape[0]
    ri = jax.lax.broadcasted_iota(jnp.int32, (r, r), 0)
    cj = jax.lax.broadcasted_iota(jnp.int32, (r, r), 1)
    fold = ((ri % (r // 4)) == (cj % (r // 4))).astype(jnp.float32)
    stats = jnp.dot(fold, jnp.concatenate([s_acc, q_acc], axis=1),
                    preferred_element_type=jnp.float32)
    cnt = float(n_batch * 16 * S)
    mean = stats[:, 0:1] * (1.0 / cnt)
    var = jnp.maximum(stats[:, 1:2] * (1.0 / cnt) - mean * mean, 0.0)
    scale = g_ref[...] * jax.lax.rsqrt(var + eps)
    shift = b_ref[...] - mean * scale
    for n in range(n_batch):
        yb = jnp.maximum(ys[n] * scale + shift, 0.0).astype(o_ref.dtype)
        for p in range(4):
            o_ref[n, p] = yb[:, p * S:(p + 1) * S]


def _ct6_kernel(x_ref, w_ref, o_ref, *, H, W):
    S = H * W
    col = jax.lax.broadcasted_iota(jnp.int32, (1, S), 1)
    aa = col // W
    bb = col % W
    C = x_ref.shape[2] // 4
    w = w_ref[...]
    piece_patches = []
    for py in range(2):
        for px in range(2):
            for py2 in range(2):
                for px2 in range(2):
                    parts = []
                    for dy in (-1, 0, 1):
                        for dx in (-1, 0, 1):
                            ty = 2 * py + py2 + dy
                            tx = 2 * px + px2 + dx
                            qy, qx = ty % 4, tx % 4
                            sy, sx = (ty - qy) // 4, (tx - qx) // 4
                            blk = x_ref[0, 2 * (qy // 2) + (qx // 2),
                                        (2 * (qy % 2) + (qx % 2)) * C:
                                        (2 * (qy % 2) + (qx % 2) + 1) * C]
                            shifted = _lane_shift(blk, sy * W + sx)
                            valid = ((aa + sy >= 0) & (aa + sy < H) &
                                     (bb + sx >= 0) & (bb + sx < W))
                            parts.append(jnp.where(valid, shifted, 0.0))
                    piece_patches.append(jnp.concatenate(parts, axis=0))
    patches = jnp.concatenate(piece_patches, axis=1)
    y = jnp.dot(w, patches, preferred_element_type=jnp.float32)
    yb = y.astype(o_ref.dtype)
    idx = 0
    for py in range(2):
        for px in range(2):
            for py2 in range(2):
                for px2 in range(2):
                    o_ref[0, 2 * py + px, 2 * py2 + px2] = \
                        yb[:, idx * S:(idx + 1) * S]
                    idx += 1


def _tail_kernel(x_ref, n1_ref, n2_ref, w1_ref, b1_ref, w2_ref, b2_ref,
                 w3_ref, b3_ref, w4_ref, b4_ref, o_ref, *, H, W, nc,
                 upper, lower):
    S = H * W
    c2 = n2_ref.shape[1]
    for i in range(x_ref.shape[0]):
        a = x_ref[i].astype(jnp.float32)
        n1 = n1_ref[i].astype(jnp.float32)
        a = _gen_noise(a, n1, upper, lower)
        a = _conv3x3(a, w1_ref[...], b1_ref[...], H, W)
        a = _conv3x3(a, w2_ref[...], b2_ref[...], H, W)
        n2 = jnp.concatenate(
            [n2_ref[i].astype(jnp.float32),
             jnp.zeros((a.shape[0] - c2, S), jnp.float32)], axis=0)
        a = _gen_noise(a, n2, upper, lower)
        a = _conv3x3(a, w3_ref[...], b3_ref[...], H, W)
        a = _conv3x3(a, w4_ref[...], b4_ref[...], H, W)
        o_ref[i] = jnp.tanh(a[:nc, :])


def _head(a1, w1, g1, b1, w2, g2, b2, w3, g3, b3):
    M = a1.shape[0]

    def rep(arr):
        return pl.BlockSpec(arr.shape, lambda i, nd=arr.ndim: (0,) * nd)

    return pl.pallas_call(
        functools.partial(_head_kernel, eps=_EPS),
        out_shape=jax.ShapeDtypeStruct((4, M, 256), jnp.bfloat16),
        grid=(1,),
        in_specs=[rep(a1), rep(w1), rep(g1), rep(b1),
                  rep(w2), rep(g2), rep(b2), rep(w3), rep(g3), rep(b3)],
        out_specs=pl.BlockSpec((4, M, 256), lambda i: (0, 0, 0)),
        compiler_params=pltpu.CompilerParams(
            dimension_semantics=("arbitrary",),
            vmem_limit_bytes=_VMEM_LIMIT),
    )(a1.astype(jnp.bfloat16), w1, g1, b1, w2, g2, b2, w3, g3, b3)


def _mid(x, w4, g4r, b4r, wpl, gamma_rows, beta_rows, *, H, W):
    N, Cin, S = x.shape
    R4, K4 = w4.shape
    R, K = wpl.shape
    return pl.pallas_call(
        functools.partial(_mid_kernel, H=H, W=W, eps=_EPS),
        out_shape=jax.ShapeDtypeStruct((N, 4, R, S), jnp.bfloat16),
        grid=(1,),
        in_specs=[pl.BlockSpec((N, Cin, S), lambda i: (0, 0, 0)),
                  pl.BlockSpec((R4, K4), lambda i: (0, 0)),
                  pl.BlockSpec((R4, 1), lambda i: (0, 0)),
                  pl.BlockSpec((R4, 1), lambda i: (0, 0)),
                  pl.BlockSpec((R, K), lambda i: (0, 0)),
                  pl.BlockSpec((R, 1), lambda i: (0, 0)),
                  pl.BlockSpec((R, 1), lambda i: (0, 0))],
        out_specs=pl.BlockSpec((N, 4, R, S), lambda i: (0, 0, 0, 0)),
        compiler_params=pltpu.CompilerParams(
            dimension_semantics=("arbitrary",),
            vmem_limit_bytes=_VMEM_LIMIT),
    )(x, w4, g4r, b4r, wpl, gamma_rows, beta_rows)


def _ct6_phase(x, wpl, *, H, W):
    N, P4, R5, S = x.shape
    R, K = wpl.shape
    return pl.pallas_call(
        functools.partial(_ct6_kernel, H=H, W=W),
        out_shape=jax.ShapeDtypeStruct((N, 4, 4, R, S), jnp.bfloat16),
        grid=(N,),
        in_specs=[pl.BlockSpec((1, P4, R5, S), lambda n: (n, 0, 0, 0)),
                  pl.BlockSpec((R, K), lambda n: (0, 0))],
        out_specs=pl.BlockSpec((1, 4, 4, R, S), lambda n: (n, 0, 0, 0, 0)),
        compiler_params=pltpu.CompilerParams(
            dimension_semantics=("parallel",),
            vmem_limit_bytes=_VMEM_LIMIT),
    )(x, wpl)


def _tail(act, n1, n2, weights, *, nc, H, W, nb=2, upper=4.0, lower=2.0):
    N, C0, S = act.shape
    c2 = n2.shape[1]
    w1, b1, w2, b2, w3, b3, w4, b4 = weights

    def rep_spec(arr):
        nd = arr.ndim
        return pl.BlockSpec(arr.shape, lambda n, nd=nd: (0,) * nd)

    return pl.pallas_call(
        functools.partial(_tail_kernel, H=H, W=W, nc=nc,
                          upper=upper, lower=lower),
        out_shape=jax.ShapeDtypeStruct((N, nc, S), jnp.float32),
        grid=(N // nb,),
        in_specs=[pl.BlockSpec((nb, C0, S), lambda n: (n, 0, 0)),
                  pl.BlockSpec((nb, C0, S), lambda n: (n, 0, 0)),
                  pl.BlockSpec((nb, c2, S), lambda n: (n, 0, 0)),
                  rep_spec(w1), rep_spec(b1), rep_spec(w2), rep_spec(b2),
                  rep_spec(w3), rep_spec(b3), rep_spec(w4), rep_spec(b4)],
        out_specs=pl.BlockSpec((nb, nc, S), lambda n: (n, 0, 0)),
        compiler_params=pltpu.CompilerParams(
            dimension_semantics=("parallel",),
            vmem_limit_bytes=_VMEM_LIMIT),
    )(act, n1, n2, w1, b1, w2, b2, w3, b3, w4, b4)


def kernel(m1, m2, m3, m4, m5, m6,
           g1, b1, g2, b2, g3, b3, g4, b4, g5, b5,
           c1_w, c1_b, c2_w, c2_b, c3_w, c3_b, c4_w, c4_b,
           x, noise1, noise2):
    nc, ngf = 1, 16
    N, nz = x.shape[0], x.shape[1]
    z = x.reshape(N, nz).astype(jnp.bfloat16)

    eye16 = jnp.eye(16, dtype=z.dtype)
    a1 = (eye16[None, :, :, None] * z[:, None, None, :]).reshape(
        N * 16, 16 * nz)
    y3 = _head(a1, m1, g1, b1, m2, g2, b2, m3, g3, b3)
    hp = (y3.reshape(2, 2, N, 4, 4, 2, 2, 64)
          .transpose(2, 7, 3, 0, 5, 4, 1, 6)
          .reshape(N, ngf * 4, 256))

    m4pm = m4.reshape(ngf * 2, 4, m4.shape[1]).transpose(1, 0, 2) \
        .reshape(ngf * 8, m4.shape[1])
    g4pm = g4.reshape(ngf * 2, 4).transpose(1, 0).reshape(ngf * 8, 1)
    b4pm = b4.reshape(ngf * 2, 4).transpose(1, 0).reshape(ngf * 8, 1)
    m5pm = m5.reshape(ngf, 4, m5.shape[1]).transpose(1, 0, 2) \
        .reshape(ngf * 4, m5.shape[1])
    g5pm = g5.reshape(ngf, 4).transpose(1, 0).reshape(ngf * 4, 1)
    b5pm = b5.reshape(ngf, 4).transpose(1, 0).reshape(ngf * 4, 1)
    y5p = _mid(hp, m4pm, g4pm, b4pm, m5pm, g5pm, b5pm,
               H=16, W=16)

    y6 = _ct6_phase(y5p, m6, H=16, W=16)
    act = (y6.reshape(N, 2, 2, 2, 2, 8, 2, 2, 16, 16)
           .transpose(0, 5, 8, 1, 3, 6, 9, 2, 4, 7)
           .reshape(N, ngf // 2, 128 * 128))

    S = 128 * 128
    n1 = noise1.reshape(N, ngf // 2, S)
    n2 = noise2.reshape(N, noise2.shape[1], S)
    weights = (c1_w, c1_b, c2_w, c2_b, c3_w, c3_b, c4_w, c4_b)
    out = _tail(act, n1, n2, weights, nc=nc, H=128, W=128)
    return out.reshape(N, nc, 128, 128)


# speedup vs baseline: 1.1577x; 1.0149x over previous
import functools

import jax
import jax.numpy as jnp
from jax.experimental import pallas as pl
from jax.experimental.pallas import tpu as pltpu

_EPS = 1e-5
_VMEM_LIMIT = 48 * 1024 * 1024


def _lane_shift(x, d):
    S = x.shape[-1]
    d = d % S
    if d == 0:
        return x
    return jnp.concatenate([x[:, d:], x[:, :d]], axis=-1)


def _gather3x3(x, H, W):
    S = H * W
    col = jax.lax.broadcasted_iota(jnp.int32, (1, S), 1)
    yy = col // W
    xx = col % W
    parts = []
    for wy in range(3):
        for wx in range(3):
            dy, dx = wy - 1, wx - 1
            shifted = _lane_shift(x, dy * W + dx)
            valid = ((yy + dy >= 0) & (yy + dy < H) &
                     (xx + dx >= 0) & (xx + dx < W))
            parts.append(jnp.where(valid, shifted, 0.0))
    return jnp.concatenate(parts, axis=0)


def _gen_noise(x, noise, upper, lower):
    S = x.shape[-1]
    cmax = jnp.max(x, axis=-1, keepdims=True)
    s = jnp.sum(x, axis=-1, keepdims=True)
    q = jnp.sum(x * x, axis=-1, keepdims=True)
    mean = s * (1.0 / S)
    var = jnp.maximum((q - S * mean * mean) * (1.0 / (S - 1)), 0.0)
    std = jnp.sqrt(var)
    clone = jnp.where(x < -cmax * (1.0 / lower), 0.0, x)
    clone = jnp.where(clone > cmax * (1.0 / upper), 0.0, clone)
    return x + clone * (noise * std)


def _conv3x3(a, wmat, bias, H, W):
    patches = _gather3x3(a.astype(jnp.bfloat16), H, W)
    y = jnp.dot(wmat, patches, preferred_element_type=jnp.float32)
    return y + bias


def _row_shift(x, d):
    if d % x.shape[0] == 0:
        return x
    return jnp.concatenate([x[d:], x[:d]], axis=0)


def _bn_cols_folded(y, g, b, cout, eps):
    m = y.shape[0]
    s = jnp.sum(y, axis=0, keepdims=True)
    q = jnp.sum(y * y, axis=0, keepdims=True)
    sc = s[:, 0:cout] + s[:, cout:2 * cout] + s[:, 2 * cout:3 * cout] \
        + s[:, 3 * cout:4 * cout]
    qc = q[:, 0:cout] + q[:, cout:2 * cout] + q[:, 2 * cout:3 * cout] \
        + q[:, 3 * cout:4 * cout]
    cnt = 4.0 * m
    mean = sc * (1.0 / cnt)
    var = jnp.maximum(qc * (1.0 / cnt) - mean * mean, 0.0)
    scale = g * jax.lax.rsqrt(var + eps)
    shift = b - mean * scale
    scale4 = jnp.concatenate([scale] * 4, axis=1)
    shift4 = jnp.concatenate([shift] * 4, axis=1)
    return jnp.maximum(y * scale4 + shift4, 0.0)


def _head_kernel(a_ref, w1_ref, g1_ref, b1_ref, w2_ref, g2_ref, b2_ref,
                 w3_ref, g3_ref, b3_ref, o_ref, *, eps):
    y = jnp.dot(a_ref[...], w1_ref[...], preferred_element_type=jnp.float32)
    m = y.shape[0]
    mean = jnp.sum(y, axis=0, keepdims=True) * (1.0 / m)
    var = jnp.maximum(jnp.sum(y * y, axis=0, keepdims=True) * (1.0 / m)
                      - mean * mean, 0.0)
    scale = g1_ref[...] * jax.lax.rsqrt(var + eps)
    shift = b1_ref[...] - mean * scale
    h1 = jnp.maximum(y * scale + shift, 0.0).astype(jnp.bfloat16)

    row = jax.lax.broadcasted_iota(jnp.int32, (m, 1), 0)
    yy = (row % 16) // 4
    xx = row % 4
    parts = []
    for dy in (-1, 0, 1):
        for dx in (-1, 0, 1):
            shifted = _row_shift(h1, dy * 4 + dx)
            valid = ((yy + dy >= 0) & (yy + dy < 4) &
                     (xx + dx >= 0) & (xx + dx < 4))
            parts.append(jnp.where(valid, shifted, 0.0))
    patches2 = jnp.concatenate(parts, axis=1)
    y2 = jnp.dot(patches2, w2_ref[...], preferred_element_type=jnp.float32)
    h2 = _bn_cols_folded(y2, g2_ref[...], b2_ref[...], 128, eps) \
        .astype(jnp.bfloat16)

    aa = (row % 16) // 4
    bb = row % 4
    y3s = {}
    s_acc = None
    q_acc = None
    for ry in range(2):
        for rx in range(2):
            p3 = []
            for dy in (-1, 0, 1):
                for dx in (-1, 0, 1):
                    ty, tx = ry + dy, rx + dx
                    qy, qx = ty % 2, tx % 2
                    sy, sx = (ty - qy) // 2, (tx - qx) // 2
                    g = (2 * qy + qx) * 128
                    blk = _row_shift(h2[:, g:g + 128], sy * 4 + sx)
                    valid = ((aa + sy >= 0) & (aa + sy < 4) &
                             (bb + sx >= 0) & (bb + sx < 4))
                    p3.append(jnp.where(valid, blk, 0.0))
            patches3 = jnp.concatenate(p3, axis=1)
            y3 = jnp.dot(patches3, w3_ref[...],
                         preferred_element_type=jnp.float32)
            y3s[(ry, rx)] = y3
            s = jnp.sum(y3, axis=0, keepdims=True)
            q = jnp.sum(y3 * y3, axis=0, keepdims=True)
            s_acc = s if s_acc is None else s_acc + s
            q_acc = q if q_acc is None else q_acc + q
    cout = 64
    sc = s_acc[:, 0:cout] + s_acc[:, cout:2 * cout] \
        + s_acc[:, 2 * cout:3 * cout] + s_acc[:, 3 * cout:4 * cout]
    qc = q_acc[:, 0:cout] + q_acc[:, cout:2 * cout] \
        + q_acc[:, 2 * cout:3 * cout] + q_acc[:, 3 * cout:4 * cout]
    cnt = 16.0 * m
    mean = sc * (1.0 / cnt)
    var = jnp.maximum(qc * (1.0 / cnt) - mean * mean, 0.0)
    scale = g3_ref[...] * jax.lax.rsqrt(var + eps)
    shift = b3_ref[...] - mean * scale
    scale4 = jnp.concatenate([scale] * 4, axis=1)
    shift4 = jnp.concatenate([shift] * 4, axis=1)
    for p, (ry, rx) in enumerate([(0, 0), (0, 1), (1, 0), (1, 1)]):
        o_ref[p] = jnp.maximum(y3s[(ry, rx)] * scale4 + shift4,
                               0.0).astype(o_ref.dtype)


def _mid_kernel(x_ref, w4_ref, g4_ref, b4_ref, w_ref, g_ref, b_ref, o_ref,
                *, H, W, eps):
    n_batch = x_ref.shape[0]
    S = H * W
    w4 = w4_ref[...]
    y4s, s4_acc, q4_acc = [], None, None
    for n in range(n_batch):
        patches = _gather3x3(x_ref[n], H, W)
        y4 = jnp.dot(w4, patches, preferred_element_type=jnp.float32)
        y4s.append(y4)
        s_n = jnp.sum(y4, axis=1, keepdims=True)
        q_n = jnp.sum(y4 * y4, axis=1, keepdims=True)
        s4_acc = s_n if s4_acc is None else s4_acc + s_n
        q4_acc = q_n if q4_acc is None else q4_acc + q_n
    r4 = s4_acc.shape[0]
    ri = jax.lax.broadcasted_iota(jnp.int32, (r4, r4), 0)
    cj = jax.lax.broadcasted_iota(jnp.int32, (r4, r4), 1)
    fold4 = ((ri % (r4 // 4)) == (cj % (r4 // 4))).astype(jnp.float32)
    st4 = jnp.dot(fold4, jnp.concatenate([s4_acc, q4_acc], axis=1),
                  preferred_element_type=jnp.float32)
    cnt4 = float(n_batch * 4 * S)
    mean4 = st4[:, 0:1] * (1.0 / cnt4)
    var4 = jnp.maximum(st4[:, 1:2] * (1.0 / cnt4) - mean4 * mean4, 0.0)
    scale4 = g4_ref[...] * jax.lax.rsqrt(var4 + eps)
    shift4 = b4_ref[...] - mean4 * scale4

    C = r4 // 4
    w = w_ref[...]
    col = jax.lax.broadcasted_iota(jnp.int32, (1, S), 1)
    aa = col // W
    bb = col % W
    ys, s_acc, q_acc = [], None, None
    for n in range(n_batch):
        x = jnp.maximum(y4s[n] * scale4 + shift4, 0.0).astype(jnp.bfloat16)
        piece_patches = []
        for py in range(2):
            for px in range(2):
                parts = []
                for dy in (-1, 0, 1):
                    for dx in (-1, 0, 1):
                        ty, tx = py + dy, px + dx
                        qy, qx = ty % 2, tx % 2
                        sy, sx = (ty - qy) // 2, (tx - qx) // 2
                        blk = x[(2 * qy + qx) * C:(2 * qy + qx + 1) * C]
                        shifted = _lane_shift(blk, sy * W + sx)
                        valid = ((aa + sy >= 0) & (aa + sy < H) &
                                 (bb + sx >= 0) & (bb + sx < W))
                        parts.append(jnp.where(valid, shifted, 0.0))
                piece_patches.append(jnp.concatenate(parts, axis=0))
        patches = jnp.concatenate(piece_patches, axis=1)
        y = jnp.dot(w, patches, preferred_element_type=jnp.float32)
        ys.append(y)
        s_p = jnp.sum(y, axis=1, keepdims=True)
        q_p = jnp.sum(y * y, axis=1, keepdims=True)
        s_acc = s_p if s_acc is None else s_acc + s_p
        q_acc = q_p if q_acc is None else q_acc + q_p
    r = s_acc.shape[0]
    ri = jax.lax.broadcasted_iota(jnp.int32, (r, r), 0)
    cj = jax.lax.broadcasted_iota(jnp.int32, (r, r), 1)
    fold = ((ri % (r // 4)) == (cj % (r // 4))).astype(jnp.float32)
    stats = jnp.dot(fold, jnp.concatenate([s_acc, q_acc], axis=1),
                    preferred_element_type=jnp.float32)
    cnt = float(n_batch * 16 * S)
    mean = stats[:, 0:1] * (1.0 / cnt)
    var = jnp.maximum(stats[:, 1:2] * (1.0 / cnt) - mean * mean, 0.0)
    scale = g_ref[...] * jax.lax.rsqrt(var + eps)
    shift = b_ref[...] - mean * scale
    for n in range(n_batch):
        yb = jnp.maximum(ys[n] * scale + shift, 0.0).astype(o_ref.dtype)
        for p in range(4):
            o_ref[n, p] = yb[:, p * S:(p + 1) * S]


def _ct6_kernel(x_ref, w_ref, o_ref, *, H, W):
    S = H * W
    col = jax.lax.broadcasted_iota(jnp.int32, (1, S), 1)
    aa = col // W
    bb = col % W
    C = x_ref.shape[2] // 4
    w = w_ref[...]
    piece_patches = []
    for py in range(2):
        for px in range(2):
            for py2 in range(2):
                for px2 in range(2):
                    parts = []
                    for dy in (-1, 0, 1):
                        for dx in (-1, 0, 1):
                            ty = 2 * py + py2 + dy
                            tx = 2 * px + px2 + dx
                            qy, qx = ty % 4, tx % 4
                            sy, sx = (ty - qy) // 4, (tx - qx) // 4
                            blk = x_ref[0, 2 * (qy // 2) + (qx // 2),
                                        (2 * (qy % 2) + (qx % 2)) * C:
                                        (2 * (qy % 2) + (qx % 2) + 1) * C]
                            shifted = _lane_shift(blk, sy * W + sx)
                            valid = ((aa + sy >= 0) & (aa + sy < H) &
                                     (bb + sx >= 0) & (bb + sx < W))
                            parts.append(jnp.where(valid, shifted, 0.0))
                    piece_patches.append(jnp.concatenate(parts, axis=0))
    patches = jnp.concatenate(piece_patches, axis=1)
    y = jnp.dot(w, patches, preferred_element_type=jnp.float32)
    yb = y.astype(o_ref.dtype)
    idx = 0
    for py in range(2):
        for px in range(2):
            for py2 in range(2):
                for px2 in range(2):
                    o_ref[0, 2 * py + px, 2 * py2 + px2] = \
                        yb[:, idx * S:(idx + 1) * S]
                    idx += 1


def _tail_kernel(x_ref, n1_ref, n2_ref, w1_ref, b1_ref, w2_ref, b2_ref,
                 w3_ref, b3_ref, w4_ref, b4_ref, o_ref, *, H, W, nc,
                 upper, lower):
    S = H * W
    c2 = n2_ref.shape[1]
    for i in range(x_ref.shape[0]):
        a = x_ref[i].astype(jnp.float32)
        n1 = n1_ref[i].astype(jnp.float32)
        a = _gen_noise(a, n1, upper, lower)
        a = _conv3x3(a, w1_ref[...], b1_ref[...], H, W)
        a = _conv3x3(a, w2_ref[...], b2_ref[...], H, W)
        n2 = jnp.concatenate(
            [n2_ref[i].astype(jnp.float32),
             jnp.zeros((a.shape[0] - c2, S), jnp.float32)], axis=0)
        a = _gen_noise(a, n2, upper, lower)
        a = _conv3x3(a, w3_ref[...], b3_ref[...], H, W)
        a = _conv3x3(a, w4_ref[...], b4_ref[...], H, W)
        o_ref[i] = jnp.tanh(a[:nc, :])


def _head(a1, w1, g1, b1, w2, g2, b2, w3, g3, b3):
    M = a1.shape[0]

    def rep(arr):
        return pl.BlockSpec(arr.shape, lambda i, nd=arr.ndim: (0,) * nd)

    return pl.pallas_call(
        functools.partial(_head_kernel, eps=_EPS),
        out_shape=jax.ShapeDtypeStruct((4, M, 256), jnp.bfloat16),
        grid=(1,),
        in_specs=[rep(a1), rep(w1), rep(g1), rep(b1),
                  rep(w2), rep(g2), rep(b2), rep(w3), rep(g3), rep(b3)],
        out_specs=pl.BlockSpec((4, M, 256), lambda i: (0, 0, 0)),
        compiler_params=pltpu.CompilerParams(
            dimension_semantics=("arbitrary",),
            vmem_limit_bytes=_VMEM_LIMIT),
    )(a1.astype(jnp.bfloat16), w1, g1, b1, w2, g2, b2, w3, g3, b3)


def _mid(x, w4, g4r, b4r, wpl, gamma_rows, beta_rows, *, H, W):
    N, Cin, S = x.shape
    R4, K4 = w4.shape
    R, K = wpl.shape
    return pl.pallas_call(
        functools.partial(_mid_kernel, H=H, W=W, eps=_EPS),
        out_shape=jax.ShapeDtypeStruct((N, 4, R, S), jnp.bfloat16),
        grid=(1,),
        in_specs=[pl.BlockSpec((N, Cin, S), lambda i: (0, 0, 0)),
                  pl.BlockSpec((R4, K4), lambda i: (0, 0)),
                  pl.BlockSpec((R4, 1), lambda i: (0, 0)),
                  pl.BlockSpec((R4, 1), lambda i: (0, 0)),
                  pl.BlockSpec((R, K), lambda i: (0, 0)),
                  pl.BlockSpec((R, 1), lambda i: (0, 0)),
                  pl.BlockSpec((R, 1), lambda i: (0, 0))],
        out_specs=pl.BlockSpec((N, 4, R, S), lambda i: (0, 0, 0, 0)),
        compiler_params=pltpu.CompilerParams(
            dimension_semantics=("arbitrary",),
            vmem_limit_bytes=_VMEM_LIMIT),
    )(x, w4, g4r, b4r, wpl, gamma_rows, beta_rows)


def _ct6_phase(x, wpl, *, H, W):
    N, P4, R5, S = x.shape
    R, K = wpl.shape
    return pl.pallas_call(
        functools.partial(_ct6_kernel, H=H, W=W),
        out_shape=jax.ShapeDtypeStruct((N, 4, 4, R, S), jnp.bfloat16),
        grid=(N,),
        in_specs=[pl.BlockSpec((1, P4, R5, S), lambda n: (n, 0, 0, 0)),
                  pl.BlockSpec((R, K), lambda n: (0, 0))],
        out_specs=pl.BlockSpec((1, 4, 4, R, S), lambda n: (n, 0, 0, 0, 0)),
        compiler_params=pltpu.CompilerParams(
            dimension_semantics=("parallel",),
            vmem_limit_bytes=_VMEM_LIMIT),
    )(x, wpl)


def _tail(act, n1, n2, weights, *, nc, H, W, nb=4, upper=4.0, lower=2.0):
    N, C0, S = act.shape
    c2 = n2.shape[1]
    w1, b1, w2, b2, w3, b3, w4, b4 = weights

    def rep_spec(arr):
        nd = arr.ndim
        return pl.BlockSpec(arr.shape, lambda n, nd=nd: (0,) * nd)

    return pl.pallas_call(
        functools.partial(_tail_kernel, H=H, W=W, nc=nc,
                          upper=upper, lower=lower),
        out_shape=jax.ShapeDtypeStruct((N, nc, S), jnp.float32),
        grid=(N // nb,),
        in_specs=[pl.BlockSpec((nb, C0, S), lambda n: (n, 0, 0)),
                  pl.BlockSpec((nb, C0, S), lambda n: (n, 0, 0)),
                  pl.BlockSpec((nb, c2, S), lambda n: (n, 0, 0)),
                  rep_spec(w1), rep_spec(b1), rep_spec(w2), rep_spec(b2),
                  rep_spec(w3), rep_spec(b3), rep_spec(w4), rep_spec(b4)],
        out_specs=pl.BlockSpec((nb, nc, S), lambda n: (n, 0, 0)),
        compiler_params=pltpu.CompilerParams(
            dimension_semantics=("parallel",),
            vmem_limit_bytes=_VMEM_LIMIT),
    )(act, n1, n2, w1, b1, w2, b2, w3, b3, w4, b4)


def kernel(m1, m2, m3, m4, m5, m6,
           g1, b1, g2, b2, g3, b3, g4, b4, g5, b5,
           c1_w, c1_b, c2_w, c2_b, c3_w, c3_b, c4_w, c4_b,
           x, noise1, noise2):
    nc, ngf = 1, 16
    N, nz = x.shape[0], x.shape[1]
    z = x.reshape(N, nz).astype(jnp.bfloat16)

    eye16 = jnp.eye(16, dtype=z.dtype)
    a1 = (eye16[None, :, :, None] * z[:, None, None, :]).reshape(
        N * 16, 16 * nz)
    y3 = _head(a1, m1, g1, b1, m2, g2, b2, m3, g3, b3)
    hp = (y3.reshape(2, 2, N, 4, 4, 2, 2, 64)
          .transpose(2, 7, 3, 0, 5, 4, 1, 6)
          .reshape(N, ngf * 4, 256))

    m4pm = m4.reshape(ngf * 2, 4, m4.shape[1]).transpose(1, 0, 2) \
        .reshape(ngf * 8, m4.shape[1])
    g4pm = g4.reshape(ngf * 2, 4).transpose(1, 0).reshape(ngf * 8, 1)
    b4pm = b4.reshape(ngf * 2, 4).transpose(1, 0).reshape(ngf * 8, 1)
    m5pm = m5.reshape(ngf, 4, m5.shape[1]).transpose(1, 0, 2) \
        .reshape(ngf * 4, m5.shape[1])
    g5pm = g5.reshape(ngf, 4).transpose(1, 0).reshape(ngf * 4, 1)
    b5pm = b5.reshape(ngf, 4).transpose(1, 0).reshape(ngf * 4, 1)
    y5p = _mid(hp, m4pm, g4pm, b4pm, m5pm, g5pm, b5pm,
               H=16, W=16)

    y6 = _ct6_phase(y5p, m6, H=16, W=16)
    act = (y6.reshape(N, 2, 2, 2, 2, 8, 2, 2, 16, 16)
           .transpose(0, 5, 8, 1, 3, 6, 9, 2, 4, 7)
           .reshape(N, ngf // 2, 128 * 128))

    S = 128 * 128
    n1 = noise1.reshape(N, ngf // 2, S)
    n2 = noise2.reshape(N, noise2.shape[1], S)
    weights = (c1_w, c1_b, c2_w, c2_b, c3_w, c3_b, c4_w, c4_b)
    out = _tail(act, n1, n2, weights, nc=nc, H=128, W=128)
    return out.reshape(N, nc, 128, 128)
